# Initial kernel scaffold; baseline (speedup 1.0000x reference)
import jax
import jax.numpy as jnp
from jax.experimental import pallas as pl


def kernel(x, edge_index, edge_type, W1, root1, b1, W2, root2, b2, rel_emb):
    raise NotImplementedError("write your pallas kernel here")


# speedup vs baseline: 14.5753x; 14.5753x over previous
import functools

import jax
import jax.numpy as jnp
from jax import lax
from jax.experimental import pallas as pl
from jax.experimental.pallas import tpu as pltpu
from jax.experimental.pallas import tpu_sc as plsc

N = 10000
R = 16
D = 128
E = 320000

NC = 2
NS = 16
L = 16
NW = NC * NS
EW = E // NW
C = 80
NCH = EW // C
NR = N * R
RU = 80
NU = N // RU

f32 = jnp.float32
i32 = jnp.int32

_mesh = plsc.VectorSubcoreMesh(core_axis_name="c", subcore_axis_name="s")


def _worker_id():
    return lax.axis_index("s") * NC + lax.axis_index("c")


@functools.partial(
    pl.kernel,
    out_type=jax.ShapeDtypeStruct((E,), f32),
    mesh=_mesh,
    compiler_params=pltpu.CompilerParams(needs_layout_passes=False),
    scratch_types=[
        pltpu.VMEM_SHARED((NR,), f32),
        pltpu.VMEM((2000,), f32),
        pltpu.VMEM((C,), i32),
        pltpu.VMEM((C,), i32),
        pltpu.VMEM((C,), i32),
        pltpu.VMEM((C,), f32),
        pltpu.VMEM((C,), f32),
    ],
)
def _norm_kernel(dst_hbm, typ_hbm, norm_hbm, counts_sh, stage_v, dst_v, typ_v,
                 comb_v, cnt_v, nrm_v):
    sid = lax.axis_index("s")
    wid = _worker_id()

    def fill(i, _):
        stage_v[pl.ds(i * L, L)] = jnp.zeros((L,), f32)
        return 0

    lax.fori_loop(0, 2000 // L, fill, 0)

    def zero_counts(j, _):
        pltpu.sync_copy(stage_v, counts_sh.at[pl.ds(sid * (NR // NS) + j * 2000, 2000)])
        return 0

    lax.fori_loop(0, NR // NS // 2000, zero_counts, 0)
    plsc.subcore_barrier()

    for g in range(C // L):
        stage_v[pl.ds(g * L, L)] = jnp.ones((L,), f32)

    ebase = sid * (E // NS)

    def count_chunk(i, _):
        eb = ebase + i * C
        pltpu.sync_copy(dst_hbm.at[pl.ds(eb, C)], dst_v)
        pltpu.sync_copy(typ_hbm.at[pl.ds(eb, C)], typ_v)
        for g in range(C // L):
            sl = pl.ds(g * L, L)
            comb_v[sl] = dst_v[sl] * R + typ_v[sl]
        pltpu.sync_copy(stage_v.at[pl.ds(0, C)], counts_sh.at[comb_v], add=True)
        return 0

    lax.fori_loop(0, E // NS // C, count_chunk, 0)
    plsc.subcore_barrier()

    base = wid * EW

    def norm_chunk(i, _):
        eb = base + i * C
        pltpu.sync_copy(dst_hbm.at[pl.ds(eb, C)], dst_v)
        pltpu.sync_copy(typ_hbm.at[pl.ds(eb, C)], typ_v)
        for g in range(C // L):
            sl = pl.ds(g * L, L)
            comb_v[sl] = dst_v[sl] * R + typ_v[sl]
        pltpu.sync_copy(counts_sh.at[comb_v], cnt_v)
        for g in range(C // L):
            sl = pl.ds(g * L, L)
            nrm_v[sl] = 1.0 / jnp.maximum(cnt_v[sl], 1.0)
        pltpu.sync_copy(nrm_v, norm_hbm.at[pl.ds(eb, C)])
        return 0

    lax.fori_loop(0, NCH, norm_chunk, 0)


@functools.partial(
    pl.kernel,
    out_type=jax.ShapeDtypeStruct((NC, N, D), f32),
    mesh=_mesh,
    compiler_params=pltpu.CompilerParams(needs_layout_passes=False),
    scratch_types=[
        pltpu.VMEM_SHARED((N, D), f32),
        pltpu.VMEM((RU, D), f32),
        pltpu.VMEM((C, D), f32),
        pltpu.VMEM((C,), i32),
        pltpu.VMEM((C,), i32),
        pltpu.VMEM((C,), i32),
        pltpu.VMEM((C,), i32),
        pltpu.VMEM((C,), f32),
        pltpu.SemaphoreType.DMA,
    ],
)
def _layer_kernel(y_hbm, src_hbm, typ_hbm, dst_hbm, norm_hbm, out_hbm, agg_sh,
                  bounce_v, rows_v, src_v, typ_v, dst_v, idx_v, nrm_v, sem):
    cid = lax.axis_index("c")
    sid = lax.axis_index("s")
    wid = _worker_id()

    def fill(i, _):
        for d in range(D // L):
            bounce_v[i, pl.ds(d * L, L)] = jnp.zeros((L,), f32)
        return 0

    lax.fori_loop(0, RU, fill, 0)

    n_units = jnp.where(sid < NU - (NU // NS) * NS, NU // NS + 1, NU // NS)

    def zero_agg(k, _):
        rbase = (sid + k * NS) * RU
        pltpu.sync_copy(bounce_v, agg_sh.at[pl.ds(rbase, RU)])
        return 0

    lax.fori_loop(0, n_units, zero_agg, 0)
    plsc.subcore_barrier()

    base = wid * EW

    def chunk(i, _):
        eb = base + i * C
        pltpu.sync_copy(src_hbm.at[pl.ds(eb, C)], src_v)
        pltpu.sync_copy(typ_hbm.at[pl.ds(eb, C)], typ_v)
        pltpu.sync_copy(dst_hbm.at[pl.ds(eb, C)], dst_v)
        pltpu.sync_copy(norm_hbm.at[pl.ds(eb, C)], nrm_v)
        for g in range(C // L):
            sl = pl.ds(g * L, L)
            idx_v[sl] = typ_v[sl] * N + src_v[sl]
        pltpu.async_copy(y_hbm.at[idx_v], rows_v, sem).wait()

        def scale(g, _):
            nv16 = nrm_v[pl.ds(g * L, L)]
            for j in range(L):
                e = g * L + j
                bv = jnp.full((L,), nv16[j], f32)
                for d in range(D // L):
                    sl = pl.ds(d * L, L)
                    rows_v[e, sl] = rows_v[e, sl] * bv
            return 0

        lax.fori_loop(0, C // L, scale, 0)
        pltpu.sync_copy(rows_v, agg_sh.at[dst_v], add=True)
        return 0

    lax.fori_loop(0, NCH, chunk, 0)
    plsc.subcore_barrier()

    def writeback(k, _):
        rbase = (sid + k * NS) * RU
        pltpu.sync_copy(agg_sh.at[pl.ds(rbase, RU)], bounce_v)
        pltpu.sync_copy(bounce_v, out_hbm.at[cid, pl.ds(rbase, RU)])
        return 0

    lax.fori_loop(0, n_units, writeback, 0)


@functools.partial(
    pl.kernel,
    out_type=jax.ShapeDtypeStruct((E,), f32),
    mesh=_mesh,
    compiler_params=pltpu.CompilerParams(needs_layout_passes=False),
    scratch_types=[
        pltpu.VMEM((R, D), f32),
        pltpu.VMEM((C, D), f32),
        pltpu.VMEM((C, D), f32),
        pltpu.VMEM((C,), i32),
        pltpu.VMEM((C,), i32),
        pltpu.VMEM((C,), i32),
        pltpu.VMEM((C,), f32),
        pltpu.VMEM((L * L,), f32),
        pltpu.SemaphoreType.DMA,
        pltpu.SemaphoreType.DMA,
    ],
)
def _score_kernel(h_hbm, src_hbm, dst_hbm, typ_hbm, rel_hbm, out_hbm, rel_v,
                  head_v, tail_v, src_v, dst_v, typ_v, sc_v, tp_v, sem1, sem2):
    wid = _worker_id()
    pltpu.sync_copy(rel_hbm, rel_v)
    base = wid * EW

    def chunk(i, _):
        eb = base + i * C
        pltpu.sync_copy(src_hbm.at[pl.ds(eb, C)], src_v)
        pltpu.sync_copy(dst_hbm.at[pl.ds(eb, C)], dst_v)
        pltpu.sync_copy(typ_hbm.at[pl.ds(eb, C)], typ_v)
        cp1 = pltpu.async_copy(h_hbm.at[src_v], head_v, sem1)
        cp2 = pltpu.async_copy(h_hbm.at[dst_v], tail_v, sem2)
        cp1.wait()
        cp2.wait()

        lane = lax.iota(i32, L)

        def edge_group(g, _):
            tv16 = typ_v[pl.ds(g * L, L)]
            for j in range(L):
                e = g * L + j
                te = tv16[j]
                acc = jnp.zeros((L,), f32)
                for d in range(D // L):
                    sl = pl.ds(d * L, L)
                    acc = acc + head_v[e, sl] * rel_v[te, sl] * tail_v[e, sl]
                plsc.store_scatter(tp_v, [lane * L + j], acc)
            res = jnp.zeros((L,), f32)
            for l in range(L):
                res = res + tp_v[pl.ds(l * L, L)]
            sc_v[pl.ds(g * L, L)] = res
            return 0

        lax.fori_loop(0, C // L, edge_group, 0)
        pltpu.sync_copy(sc_v, out_hbm.at[pl.ds(eb, C)])
        return 0

    lax.fori_loop(0, NCH, chunk, 0)


BN = 1000


def _mm_body(x_ref, w_ref, o_ref):
    o_ref[0] = jnp.dot(x_ref[...], w_ref[0], preferred_element_type=f32,
                       precision=lax.Precision.HIGHEST)


_mm = pl.pallas_call(
    _mm_body,
    grid=(N // BN, R + 1),
    in_specs=[
        pl.BlockSpec((BN, D), lambda nb, r: (nb, 0)),
        pl.BlockSpec((1, D, D), lambda nb, r: (r, 0, 0)),
    ],
    out_specs=pl.BlockSpec((1, BN, D), lambda nb, r: (r, nb, 0)),
    out_shape=jax.ShapeDtypeStruct((R + 1, N, D), f32),
)


def _comb_body(p_ref, y_ref, b_ref, o_ref):
    o_ref[...] = jnp.maximum(p_ref[0] + p_ref[1] + y_ref[0] + b_ref[...], 0.0)


_comb = pl.pallas_call(
    _comb_body,
    grid=(N // BN,),
    in_specs=[
        pl.BlockSpec((2, BN, D), lambda nb: (0, nb, 0)),
        pl.BlockSpec((1, BN, D), lambda nb: (R, nb, 0)),
        pl.BlockSpec((1, D), lambda nb: (0, 0)),
    ],
    out_specs=pl.BlockSpec((BN, D), lambda nb: (nb, 0)),
    out_shape=jax.ShapeDtypeStruct((N, D), f32),
)


def kernel(x, edge_index, edge_type, W1, root1, b1, W2, root2, b2, rel_emb):
    src = edge_index[0]
    dst = edge_index[1]
    typ = edge_type

    norm = _norm_kernel(dst, typ)

    W1a = jnp.concatenate([W1, root1[None]], axis=0)
    y1 = _mm(x, W1a)
    p1 = _layer_kernel(y1.reshape((R + 1) * N, D), src, typ, dst, norm)
    h1 = _comb(p1, y1, b1.reshape(1, D))

    W2a = jnp.concatenate([W2, root2[None]], axis=0)
    y2 = _mm(h1, W2a)
    p2 = _layer_kernel(y2.reshape((R + 1) * N, D), src, typ, dst, norm)
    h2 = _comb(p2, y2, b2.reshape(1, D))

    return _score_kernel(h2, src, dst, typ, rel_emb)


# speedup vs baseline: 15.2524x; 1.0465x over previous
import functools

import jax
import jax.numpy as jnp
from jax import lax
from jax.experimental import pallas as pl
from jax.experimental.pallas import tpu as pltpu
from jax.experimental.pallas import tpu_sc as plsc

N = 10000
R = 16
D = 128
E = 320000

NC = 2
NS = 16
L = 16
NW = NC * NS
C = 128
EP = 327680
NCH = EP // NW // C
NCC = EP // NS // C
NR2 = 161280
ZS = NR2 // NS
RU = 80
NU = N // RU

f32 = jnp.float32
i32 = jnp.int32

_mesh = plsc.VectorSubcoreMesh(core_axis_name="c", subcore_axis_name="s")
_params = pltpu.CompilerParams(needs_layout_passes=False)


def _worker_id():
    return lax.axis_index("s") * NC + lax.axis_index("c")


@functools.partial(
    pl.kernel,
    out_type=[
        jax.ShapeDtypeStruct((EP,), f32),
        jax.ShapeDtypeStruct((EP,), i32),
    ],
    mesh=_mesh,
    compiler_params=_params,
    scratch_types=[
        pltpu.VMEM_SHARED((NR2,), f32),
        pltpu.VMEM((2048,), f32),
        [pltpu.VMEM((C,), i32)] * 4,
        [pltpu.VMEM((C,), i32)] * 4,
        [pltpu.VMEM((C,), i32)] * 4,
        [pltpu.VMEM((C,), i32)] * 4,
        [pltpu.VMEM((C,), f32)] * 4,
        [pltpu.VMEM((C,), f32)] * 2,
        [pltpu.VMEM((C,), i32)] * 2,
        [pltpu.SemaphoreType.DMA] * 4,
        [pltpu.SemaphoreType.DMA] * 4,
        [pltpu.SemaphoreType.DMA] * 2,
    ],
)
def _norm_kernel(dst_hbm, typ_hbm, src_hbm, norm_hbm, gidx_hbm, counts_sh,
                 stage_v, dstr, typr, srcr, combr, cntr, nor, gor, msem, csem,
                 wsem):
    sid = lax.axis_index("s")
    wid = _worker_id()

    def fill(i, _):
        stage_v[pl.ds(i * L, L)] = jnp.zeros((L,), f32)
        return 0

    lax.fori_loop(0, 2048 // L, fill, 0)

    def zero_counts(j, _):
        pltpu.sync_copy(stage_v.at[pl.ds(0, 2016)],
                        counts_sh.at[pl.ds(sid * ZS + j * 2016, 2016)])
        return 0

    lax.fori_loop(0, ZS // 2016, zero_counts, 0)
    plsc.subcore_barrier()

    for g in range(C // L):
        stage_v[pl.ds(g * L, L)] = jnp.ones((L,), f32)
    ones = stage_v.at[pl.ds(0, C)]

    cbase = sid * NCC

    def start_meta2(c, q):
        eb = (cbase + c) * C
        pltpu.async_copy(dst_hbm.at[pl.ds(eb, C)], dstr[q], msem[q])
        pltpu.async_copy(typ_hbm.at[pl.ds(eb, C)], typr[q], msem[q])

    def wait_meta2(c, q):
        eb = (cbase + c) * C
        pltpu.make_async_copy(dst_hbm.at[pl.ds(eb, C)], dstr[q], msem[q]).wait()
        pltpu.make_async_copy(typ_hbm.at[pl.ds(eb, C)], typr[q], msem[q]).wait()

    def comb_compute(q):
        for g in range(C // L):
            gl = pl.ds(g * L, L)
            combr[q][gl] = dstr[q][gl] * R + typr[q][gl]

    def start_cscatter(q):
        pltpu.async_copy(ones, counts_sh.at[combr[q]], csem[q], add=True)

    def wait_cscatter(q):
        pltpu.make_async_copy(ones, counts_sh.at[combr[q]], csem[q]).wait()

    for c in range(4):
        start_meta2(c, c)
    for c in range(2):
        wait_meta2(c, c)
        comb_compute(c)
        start_cscatter(c)
    start_meta2(4, 0)
    start_meta2(5, 1)

    def count_slot(i, _):
        for j in range(4):
            c = 4 * i + j
            q2 = (j + 2) % 4

            @pl.when(c + 2 < NCC)
            def _():
                @pl.when(c >= 2)
                def _():
                    wait_cscatter(q2)

                wait_meta2(c + 2, q2)
                comb_compute(q2)
                start_cscatter(q2)

                @pl.when(c + 6 < NCC)
                def _():
                    start_meta2(c + 6, q2)
        return 0

    lax.fori_loop(0, NCC // 4, count_slot, 0)
    for q in range(4):
        wait_cscatter(q)
    plsc.subcore_barrier()

    base = wid * NCH

    def start_meta3(c, q):
        eb = (base + c) * C
        pltpu.async_copy(dst_hbm.at[pl.ds(eb, C)], dstr[q], msem[q])
        pltpu.async_copy(typ_hbm.at[pl.ds(eb, C)], typr[q], msem[q])
        pltpu.async_copy(src_hbm.at[pl.ds(eb, C)], srcr[q], msem[q])

    def wait_meta3(c, q):
        eb = (base + c) * C
        pltpu.make_async_copy(dst_hbm.at[pl.ds(eb, C)], dstr[q], msem[q]).wait()
        pltpu.make_async_copy(typ_hbm.at[pl.ds(eb, C)], typr[q], msem[q]).wait()
        pltpu.make_async_copy(src_hbm.at[pl.ds(eb, C)], srcr[q], msem[q]).wait()

    def start_cgather(q):
        pltpu.async_copy(counts_sh.at[combr[q]], cntr[q], csem[q])

    def wait_cgather(q):
        pltpu.make_async_copy(counts_sh.at[combr[q]], cntr[q], csem[q]).wait()

    def start_out(c, k):
        eb = (base + c) * C
        pltpu.async_copy(nor[k], norm_hbm.at[pl.ds(eb, C)], wsem[k])
        pltpu.async_copy(gor[k], gidx_hbm.at[pl.ds(eb, C)], wsem[k])

    def wait_out(c, k):
        eb = (base + c) * C
        pltpu.make_async_copy(nor[k], norm_hbm.at[pl.ds(eb, C)], wsem[k]).wait()
        pltpu.make_async_copy(gor[k], gidx_hbm.at[pl.ds(eb, C)], wsem[k]).wait()

    for c in range(4):
        start_meta3(c, c)
    for c in range(2):
        wait_meta3(c, c)
        comb_compute(c)
        start_cgather(c)

    def norm_slot(i, _):
        for j in range(4):
            c = 4 * i + j
            q = j
            q2 = (j + 2) % 4
            k = j % 2

            @pl.when(c + 2 < NCH)
            def _():
                wait_meta3(c + 2, q2)
                comb_compute(q2)
                start_cgather(q2)

            wait_cgather(q)

            @pl.when(c >= 2)
            def _():
                wait_out(c - 2, k)

            for g in range(C // L):
                gl = pl.ds(g * L, L)
                nor[k][gl] = 1.0 / jnp.maximum(cntr[q][gl], 1.0)
                gor[k][gl] = typr[q][gl] * N + srcr[q][gl]
            start_out(c, k)

            @pl.when(c + 4 < NCH)
            def _():
                start_meta3(c + 4, q)
        return 0

    lax.fori_loop(0, NCH // 4, norm_slot, 0)
    for c in (NCH - 2, NCH - 1):
        wait_out(c, c % 2)


@functools.partial(
    pl.kernel,
    out_type=jax.ShapeDtypeStruct((NC, N, D), f32),
    mesh=_mesh,
    compiler_params=_params,
    scratch_types=[
        pltpu.VMEM_SHARED((N + 8, D), f32),
        [pltpu.VMEM((C, D), f32)] * 2,
        [pltpu.VMEM((C,), i32)] * 4,
        [pltpu.VMEM((C,), i32)] * 4,
        [pltpu.VMEM((C,), f32)] * 4,
        [pltpu.SemaphoreType.DMA] * 4,
        [pltpu.SemaphoreType.DMA] * 2,
        [pltpu.SemaphoreType.DMA] * 2,
    ],
)
def _layer_kernel(y_hbm, gidx_hbm, dst_hbm, norm_hbm, out_hbm, agg_sh,
                  rows, gr, dr, nr, msem, gsem, ssem):
    cid = lax.axis_index("c")
    sid = lax.axis_index("s")
    wid = _worker_id()

    def fill(i, _):
        for d in range(D // L):
            rows[0][i, pl.ds(d * L, L)] = jnp.zeros((L,), f32)
        return 0

    lax.fori_loop(0, RU, fill, 0)

    n_units = jnp.where(sid < NU - (NU // NS) * NS, NU // NS + 1, NU // NS)

    def zero_agg(u, _):
        pltpu.sync_copy(rows[0].at[pl.ds(0, RU)],
                        agg_sh.at[pl.ds((sid + u * NS) * RU, RU)])
        return 0

    lax.fori_loop(0, n_units, zero_agg, 0)
    plsc.subcore_barrier()

    base = wid * NCH

    def start_meta(c, q):
        eb = (base + c) * C
        pltpu.async_copy(gidx_hbm.at[pl.ds(eb, C)], gr[q], msem[q])
        pltpu.async_copy(dst_hbm.at[pl.ds(eb, C)], dr[q], msem[q])
        pltpu.async_copy(norm_hbm.at[pl.ds(eb, C)], nr[q], msem[q])

    def wait_meta(c, q):
        eb = (base + c) * C
        pltpu.make_async_copy(gidx_hbm.at[pl.ds(eb, C)], gr[q], msem[q]).wait()
        pltpu.make_async_copy(dst_hbm.at[pl.ds(eb, C)], dr[q], msem[q]).wait()
        pltpu.make_async_copy(norm_hbm.at[pl.ds(eb, C)], nr[q], msem[q]).wait()

    def start_gather(q, k):
        pltpu.async_copy(y_hbm.at[gr[q]], rows[k], gsem[k])

    def wait_gather(q, k):
        pltpu.make_async_copy(y_hbm.at[gr[q]], rows[k], gsem[k]).wait()

    def start_scatter(q, k):
        pltpu.async_copy(rows[k], agg_sh.at[dr[q]], ssem[k], add=True)

    def wait_scatter(q, k):
        pltpu.make_async_copy(rows[k], agg_sh.at[dr[q]], ssem[k]).wait()

    def scale(q, k):
        def body(g, _):
            nv16 = nr[q][pl.ds(g * L, L)]
            for j in range(L):
                bv = jnp.full((L,), nv16[j], f32)
                for d in range(D // L):
                    sl = pl.ds(d * L, L)
                    rows[k][g * L + j, sl] = rows[k][g * L + j, sl] * bv
            return 0

        lax.fori_loop(0, C // L, body, 0)

    for c in range(3):
        start_meta(c, c)
    wait_meta(0, 0)
    start_gather(0, 0)

    def slot(i, _):
        for j in range(4):
            c = 4 * i + j
            k = j % 2
            kn = (j + 1) % 2
            q = j
            qn = (j + 1) % 4
            qp = (j + 3) % 4

            @pl.when(c + 1 < NCH)
            def _():
                @pl.when(c >= 1)
                def _():
                    wait_scatter(qp, kn)

                wait_meta(c + 1, qn)
                start_gather(qn, kn)

                @pl.when(c + 3 < NCH)
                def _():
                    start_meta(c + 3, qp)

            @pl.when(c < NCH)
            def _():
                wait_gather(q, k)
                scale(q, k)
                start_scatter(q, k)
        return 0

    lax.fori_loop(0, NCH // 4, slot, 0)
    wait_scatter((NCH - 2) % 4, (NCH - 2) % 2)
    wait_scatter((NCH - 1) % 4, (NCH - 1) % 2)
    plsc.subcore_barrier()

    def writeback(u, _):
        rbase = (sid + u * NS) * RU
        pltpu.sync_copy(agg_sh.at[pl.ds(rbase, RU)], rows[0].at[pl.ds(0, RU)])
        pltpu.sync_copy(rows[0].at[pl.ds(0, RU)],
                        out_hbm.at[cid, pl.ds(rbase, RU)])
        return 0

    lax.fori_loop(0, n_units, writeback, 0)


@functools.partial(
    pl.kernel,
    out_type=jax.ShapeDtypeStruct((EP,), f32),
    mesh=_mesh,
    compiler_params=_params,
    scratch_types=[
        pltpu.VMEM((R, D), f32),
        [pltpu.VMEM((C, D), f32)] * 2,
        [pltpu.VMEM((C, D), f32)] * 2,
        [pltpu.VMEM((C,), i32)] * 4,
        [pltpu.VMEM((C,), i32)] * 4,
        [pltpu.VMEM((C,), i32)] * 4,
        [pltpu.VMEM((C,), f32)] * 2,
        pltpu.VMEM((L * L,), f32),
        [pltpu.SemaphoreType.DMA] * 4,
        [pltpu.SemaphoreType.DMA] * 2,
        [pltpu.SemaphoreType.DMA] * 2,
        [pltpu.SemaphoreType.DMA] * 2,
    ],
)
def _score_kernel(h_hbm, src_hbm, dst_hbm, typ_hbm, rel_hbm, out_hbm, rel_v,
                  head, tail, srcr, dstr, typr, scr, tp_v, msem, hsem, tsem,
                  wsem):
    wid = _worker_id()
    pltpu.sync_copy(rel_hbm, rel_v)
    base = wid * NCH

    def start_meta(c, q):
        eb = (base + c) * C
        pltpu.async_copy(src_hbm.at[pl.ds(eb, C)], srcr[q], msem[q])
        pltpu.async_copy(dst_hbm.at[pl.ds(eb, C)], dstr[q], msem[q])
        pltpu.async_copy(typ_hbm.at[pl.ds(eb, C)], typr[q], msem[q])

    def wait_meta(c, q):
        eb = (base + c) * C
        pltpu.make_async_copy(src_hbm.at[pl.ds(eb, C)], srcr[q], msem[q]).wait()
        pltpu.make_async_copy(dst_hbm.at[pl.ds(eb, C)], dstr[q], msem[q]).wait()
        pltpu.make_async_copy(typ_hbm.at[pl.ds(eb, C)], typr[q], msem[q]).wait()

    def start_gathers(q, k):
        pltpu.async_copy(h_hbm.at[srcr[q]], head[k], hsem[k])
        pltpu.async_copy(h_hbm.at[dstr[q]], tail[k], tsem[k])

    def wait_gathers(q, k):
        pltpu.make_async_copy(h_hbm.at[srcr[q]], head[k], hsem[k]).wait()
        pltpu.make_async_copy(h_hbm.at[dstr[q]], tail[k], tsem[k]).wait()

    def start_out(c, k):
        eb = (base + c) * C
        pltpu.async_copy(scr[k], out_hbm.at[pl.ds(eb, C)], wsem[k])

    def wait_out(c, k):
        eb = (base + c) * C
        pltpu.make_async_copy(scr[k], out_hbm.at[pl.ds(eb, C)], wsem[k]).wait()

    lane = lax.iota(i32, L)

    def compute(q, k):
        def grp(g, _):
            tv16 = typr[q][pl.ds(g * L, L)]
            for j in range(L):
                e = g * L + j
                te = tv16[j]
                acc = jnp.zeros((L,), f32)
                for d in range(D // L):
                    sl = pl.ds(d * L, L)
                    acc = acc + head[k][e, sl] * rel_v[te, sl] * tail[k][e, sl]
                plsc.store_scatter(tp_v, [lane * L + j], acc)
            res = jnp.zeros((L,), f32)
            for l in range(L):
                res = res + tp_v[pl.ds(l * L, L)]
            scr[k][pl.ds(g * L, L)] = res
            return 0

        lax.fori_loop(0, C // L, grp, 0)

    for c in range(3):
        start_meta(c, c)
    wait_meta(0, 0)
    start_gathers(0, 0)

    def slot(i, _):
        for j in range(4):
            c = 4 * i + j
            k = j % 2
            kn = (j + 1) % 2
            q = j
            qn = (j + 1) % 4
            qp = (j + 3) % 4

            @pl.when(c + 1 < NCH)
            def _():
                wait_meta(c + 1, qn)
                start_gathers(qn, kn)

                @pl.when(c + 3 < NCH)
                def _():
                    start_meta(c + 3, qp)

            @pl.when(c < NCH)
            def _():
                wait_gathers(q, k)

                @pl.when(c >= 2)
                def _():
                    wait_out(c - 2, k)

                compute(q, k)
                start_out(c, k)
        return 0

    lax.fori_loop(0, NCH // 4, slot, 0)
    for c in (NCH - 2, NCH - 1):
        wait_out(c, c % 2)


BN = 1000


def _mm_body(x_ref, w_ref, o_ref):
    o_ref[0] = jnp.dot(x_ref[...], w_ref[0], preferred_element_type=f32,
                       precision=lax.Precision.HIGHEST)


_mm = pl.pallas_call(
    _mm_body,
    grid=(N // BN, R + 1),
    in_specs=[
        pl.BlockSpec((BN, D), lambda nb, r: (nb, 0)),
        pl.BlockSpec((1, D, D), lambda nb, r: (r, 0, 0)),
    ],
    out_specs=pl.BlockSpec((1, BN, D), lambda nb, r: (r, nb, 0)),
    out_shape=jax.ShapeDtypeStruct((R + 1, N, D), f32),
)


def _comb_body(p_ref, y_ref, b_ref, o_ref):
    o_ref[...] = jnp.maximum(p_ref[0] + p_ref[1] + y_ref[0] + b_ref[...], 0.0)


_comb = pl.pallas_call(
    _comb_body,
    grid=(N // BN,),
    in_specs=[
        pl.BlockSpec((2, BN, D), lambda nb: (0, nb, 0)),
        pl.BlockSpec((1, BN, D), lambda nb: (R, nb, 0)),
        pl.BlockSpec((1, D), lambda nb: (0, 0)),
    ],
    out_specs=pl.BlockSpec((BN, D), lambda nb: (nb, 0)),
    out_shape=jax.ShapeDtypeStruct((N, D), f32),
)


def kernel(x, edge_index, edge_type, W1, root1, b1, W2, root2, b2, rel_emb):
    pad = EP - E
    srcp = jnp.concatenate([edge_index[0], jnp.zeros((pad,), i32)])
    dstp = jnp.concatenate([edge_index[1], jnp.full((pad,), N, i32)])
    typp = jnp.concatenate([edge_type, jnp.zeros((pad,), i32)])

    norm, gidx = _norm_kernel(dstp, typp, srcp)

    W1a = jnp.concatenate([W1, root1[None]], axis=0)
    y1 = _mm(x, W1a)
    p1 = _layer_kernel(y1.reshape((R + 1) * N, D), gidx, dstp, norm)
    h1 = _comb(p1, y1, b1.reshape(1, D))

    W2a = jnp.concatenate([W2, root2[None]], axis=0)
    y2 = _mm(h1, W2a)
    p2 = _layer_kernel(y2.reshape((R + 1) * N, D), gidx, dstp, norm)
    h2 = _comb(p2, y2, b2.reshape(1, D))

    return _score_kernel(h2, srcp, dstp, typp, rel_emb)[:E]


# speedup vs baseline: 33.4866x; 2.1955x over previous
import functools

import jax
import jax.numpy as jnp
from jax import lax
from jax.experimental import pallas as pl
from jax.experimental.pallas import tpu as pltpu
from jax.experimental.pallas import tpu_sc as plsc

N = 10000
R = 16
D = 128
E = 320000

NC = 2
NS = 16
L = 16
NW = NC * NS
C = 128
EP = 327680
NCH = EP // NW // C
NCC = EP // NS // C
NR2 = 161280
ZS = NR2 // NS
RU = 80
NU = N // RU

f32 = jnp.float32
i32 = jnp.int32

_mesh = plsc.VectorSubcoreMesh(core_axis_name="c", subcore_axis_name="s")
_params = pltpu.CompilerParams(needs_layout_passes=False)


def _worker_id():
    return lax.axis_index("s") * NC + lax.axis_index("c")


@functools.partial(
    pl.kernel,
    out_type=[
        jax.ShapeDtypeStruct((EP,), f32),
        jax.ShapeDtypeStruct((EP,), i32),
    ],
    mesh=_mesh,
    compiler_params=_params,
    scratch_types=[
        pltpu.VMEM_SHARED((NR2,), f32),
        pltpu.VMEM((2048,), f32),
        [pltpu.VMEM((C,), i32)] * 4,
        [pltpu.VMEM((C,), i32)] * 4,
        [pltpu.VMEM((C,), i32)] * 4,
        [pltpu.VMEM((C,), i32)] * 4,
        [pltpu.VMEM((C,), f32)] * 4,
        [pltpu.VMEM((C,), f32)] * 2,
        [pltpu.VMEM((C,), i32)] * 2,
        [pltpu.SemaphoreType.DMA] * 4,
        [pltpu.SemaphoreType.DMA] * 4,
        [pltpu.SemaphoreType.DMA] * 2,
    ],
)
def _norm_kernel(dst_hbm, typ_hbm, src_hbm, norm_hbm, gidx_hbm, counts_sh,
                 stage_v, dstr, typr, srcr, combr, cntr, nor, gor, msem, csem,
                 wsem):
    sid = lax.axis_index("s")
    wid = _worker_id()

    def fill(i, _):
        stage_v[pl.ds(i * L, L)] = jnp.zeros((L,), f32)
        return 0

    lax.fori_loop(0, 2048 // L, fill, 0)

    def zero_counts(j, _):
        pltpu.sync_copy(stage_v.at[pl.ds(0, 2016)],
                        counts_sh.at[pl.ds(sid * ZS + j * 2016, 2016)])
        return 0

    lax.fori_loop(0, ZS // 2016, zero_counts, 0)
    plsc.subcore_barrier()

    for g in range(C // L):
        stage_v[pl.ds(g * L, L)] = jnp.ones((L,), f32)
    ones = stage_v.at[pl.ds(0, C)]

    cbase = sid * NCC

    def start_meta2(c, q):
        eb = (cbase + c) * C
        pltpu.async_copy(dst_hbm.at[pl.ds(eb, C)], dstr[q], msem[q])
        pltpu.async_copy(typ_hbm.at[pl.ds(eb, C)], typr[q], msem[q])

    def wait_meta2(c, q):
        eb = (cbase + c) * C
        pltpu.make_async_copy(dst_hbm.at[pl.ds(eb, C)], dstr[q], msem[q]).wait()
        pltpu.make_async_copy(typ_hbm.at[pl.ds(eb, C)], typr[q], msem[q]).wait()

    def comb_compute(q):
        for g in range(C // L):
            gl = pl.ds(g * L, L)
            combr[q][gl] = dstr[q][gl] * R + typr[q][gl]

    def start_cscatter(q):
        pltpu.async_copy(ones, counts_sh.at[combr[q]], csem[q], add=True)

    def wait_cscatter(q):
        pltpu.make_async_copy(ones, counts_sh.at[combr[q]], csem[q]).wait()

    for c in range(4):
        start_meta2(c, c)
    for c in range(2):
        wait_meta2(c, c)
        comb_compute(c)
        start_cscatter(c)
    start_meta2(4, 0)
    start_meta2(5, 1)

    def count_slot(i, _):
        for j in range(4):
            c = 4 * i + j
            q2 = (j + 2) % 4

            @pl.when(c + 2 < NCC)
            def _():
                @pl.when(c >= 2)
                def _():
                    wait_cscatter(q2)

                wait_meta2(c + 2, q2)
                comb_compute(q2)
                start_cscatter(q2)

                @pl.when(c + 6 < NCC)
                def _():
                    start_meta2(c + 6, q2)
        return 0

    lax.fori_loop(0, NCC // 4, count_slot, 0)
    for q in range(4):
        wait_cscatter(q)
    plsc.subcore_barrier()

    base = wid * NCH

    def start_meta3(c, q):
        eb = (base + c) * C
        pltpu.async_copy(dst_hbm.at[pl.ds(eb, C)], dstr[q], msem[q])
        pltpu.async_copy(typ_hbm.at[pl.ds(eb, C)], typr[q], msem[q])
        pltpu.async_copy(src_hbm.at[pl.ds(eb, C)], srcr[q], msem[q])

    def wait_meta3(c, q):
        eb = (base + c) * C
        pltpu.make_async_copy(dst_hbm.at[pl.ds(eb, C)], dstr[q], msem[q]).wait()
        pltpu.make_async_copy(typ_hbm.at[pl.ds(eb, C)], typr[q], msem[q]).wait()
        pltpu.make_async_copy(src_hbm.at[pl.ds(eb, C)], srcr[q], msem[q]).wait()

    def start_cgather(q):
        pltpu.async_copy(counts_sh.at[combr[q]], cntr[q], csem[q])

    def wait_cgather(q):
        pltpu.make_async_copy(counts_sh.at[combr[q]], cntr[q], csem[q]).wait()

    def start_out(c, k):
        eb = (base + c) * C
        pltpu.async_copy(nor[k], norm_hbm.at[pl.ds(eb, C)], wsem[k])
        pltpu.async_copy(gor[k], gidx_hbm.at[pl.ds(eb, C)], wsem[k])

    def wait_out(c, k):
        eb = (base + c) * C
        pltpu.make_async_copy(nor[k], norm_hbm.at[pl.ds(eb, C)], wsem[k]).wait()
        pltpu.make_async_copy(gor[k], gidx_hbm.at[pl.ds(eb, C)], wsem[k]).wait()

    for c in range(4):
        start_meta3(c, c)
    for c in range(2):
        wait_meta3(c, c)
        comb_compute(c)
        start_cgather(c)

    def norm_slot(i, _):
        for j in range(4):
            c = 4 * i + j
            q = j
            q2 = (j + 2) % 4
            k = j % 2

            @pl.when(c + 2 < NCH)
            def _():
                wait_meta3(c + 2, q2)
                comb_compute(q2)
                start_cgather(q2)

            wait_cgather(q)

            @pl.when(c >= 2)
            def _():
                wait_out(c - 2, k)

            for g in range(C // L):
                gl = pl.ds(g * L, L)
                nv = 1.0 / jnp.maximum(cntr[q][gl], 1.0)
                nor[k][gl] = jnp.where(dstr[q][gl] == N, 0.0, nv)
                gor[k][gl] = typr[q][gl] * N + srcr[q][gl]
            start_out(c, k)

            @pl.when(c + 4 < NCH)
            def _():
                start_meta3(c + 4, q)
        return 0

    lax.fori_loop(0, NCH // 4, norm_slot, 0)
    for c in (NCH - 2, NCH - 1):
        wait_out(c, c % 2)


@functools.partial(
    pl.kernel,
    out_type=jax.ShapeDtypeStruct((NC, N, D), f32),
    mesh=_mesh,
    compiler_params=_params,
    scratch_types=[
        pltpu.VMEM_SHARED((N + 8, D), f32),
        [pltpu.VMEM((C, D), f32)] * 2,
        [pltpu.VMEM((C,), i32)] * 4,
        [pltpu.VMEM((C,), i32)] * 4,
        [pltpu.VMEM((C,), f32)] * 4,
        [pltpu.SemaphoreType.DMA] * 4,
        [pltpu.SemaphoreType.DMA] * 2,
        [pltpu.SemaphoreType.DMA] * 2,
    ],
)
def _layer_kernel(y_hbm, gidx_hbm, dst_hbm, norm_hbm, out_hbm, agg_sh,
                  rows, gr, dr, nr, msem, gsem, ssem):
    cid = lax.axis_index("c")
    sid = lax.axis_index("s")
    wid = _worker_id()

    def fill(i, _):
        for d in range(D // L):
            rows[0][i, pl.ds(d * L, L)] = jnp.zeros((L,), f32)
        return 0

    lax.fori_loop(0, RU, fill, 0)

    n_units = jnp.where(sid < NU - (NU // NS) * NS, NU // NS + 1, NU // NS)

    def zero_agg(u, _):
        pltpu.sync_copy(rows[0].at[pl.ds(0, RU)],
                        agg_sh.at[pl.ds((sid + u * NS) * RU, RU)])
        return 0

    lax.fori_loop(0, n_units, zero_agg, 0)
    plsc.subcore_barrier()

    base = wid * NCH

    def start_meta(c, q):
        eb = (base + c) * C
        pltpu.async_copy(gidx_hbm.at[pl.ds(eb, C)], gr[q], msem[q])
        pltpu.async_copy(dst_hbm.at[pl.ds(eb, C)], dr[q], msem[q])
        pltpu.async_copy(norm_hbm.at[pl.ds(eb, C)], nr[q], msem[q])

    def wait_meta(c, q):
        eb = (base + c) * C
        pltpu.make_async_copy(gidx_hbm.at[pl.ds(eb, C)], gr[q], msem[q]).wait()
        pltpu.make_async_copy(dst_hbm.at[pl.ds(eb, C)], dr[q], msem[q]).wait()
        pltpu.make_async_copy(norm_hbm.at[pl.ds(eb, C)], nr[q], msem[q]).wait()

    def start_gather(q, k):
        pltpu.async_copy(y_hbm.at[gr[q]], rows[k], gsem[k])

    def wait_gather(q, k):
        pltpu.make_async_copy(y_hbm.at[gr[q]], rows[k], gsem[k]).wait()

    def start_scatter(q, k):
        pltpu.async_copy(rows[k], agg_sh.at[dr[q]], ssem[k], add=True)

    def wait_scatter(q, k):
        pltpu.make_async_copy(rows[k], agg_sh.at[dr[q]], ssem[k]).wait()

    def scale(q, k):
        def body(g, _):
            nv16 = nr[q][pl.ds(g * L, L)]
            for j in range(L):
                bv = jnp.full((L,), nv16[j], f32)
                for d in range(D // L):
                    sl = pl.ds(d * L, L)
                    rows[k][g * L + j, sl] = rows[k][g * L + j, sl] * bv
            return 0

        lax.fori_loop(0, C // L, body, 0)

    for c in range(3):
        start_meta(c, c)
    wait_meta(0, 0)
    start_gather(0, 0)

    def slot(i, _):
        for j in range(4):
            c = 4 * i + j
            k = j % 2
            kn = (j + 1) % 2
            q = j
            qn = (j + 1) % 4
            qp = (j + 3) % 4

            @pl.when(c + 1 < NCH)
            def _():
                @pl.when(c >= 1)
                def _():
                    wait_scatter(qp, kn)

                wait_meta(c + 1, qn)
                start_gather(qn, kn)

                @pl.when(c + 3 < NCH)
                def _():
                    start_meta(c + 3, qp)

            @pl.when(c < NCH)
            def _():
                wait_gather(q, k)
                scale(q, k)
                start_scatter(q, k)
        return 0

    lax.fori_loop(0, NCH // 4, slot, 0)
    wait_scatter((NCH - 2) % 4, (NCH - 2) % 2)
    wait_scatter((NCH - 1) % 4, (NCH - 1) % 2)
    plsc.subcore_barrier()

    def writeback(u, _):
        rbase = (sid + u * NS) * RU
        pltpu.sync_copy(agg_sh.at[pl.ds(rbase, RU)], rows[0].at[pl.ds(0, RU)])
        pltpu.sync_copy(rows[0].at[pl.ds(0, RU)],
                        out_hbm.at[cid, pl.ds(rbase, RU)])
        return 0

    lax.fori_loop(0, n_units, writeback, 0)


@functools.partial(
    pl.kernel,
    out_type=jax.ShapeDtypeStruct((EP,), f32),
    mesh=_mesh,
    compiler_params=_params,
    scratch_types=[
        pltpu.VMEM((R, D), f32),
        [pltpu.VMEM((C, D), f32)] * 2,
        [pltpu.VMEM((C, D), f32)] * 2,
        [pltpu.VMEM((C,), i32)] * 4,
        [pltpu.VMEM((C,), i32)] * 4,
        [pltpu.VMEM((C,), i32)] * 4,
        [pltpu.VMEM((C,), f32)] * 2,
        pltpu.VMEM((L * L,), f32),
        [pltpu.SemaphoreType.DMA] * 4,
        [pltpu.SemaphoreType.DMA] * 2,
        [pltpu.SemaphoreType.DMA] * 2,
        [pltpu.SemaphoreType.DMA] * 2,
    ],
)
def _score_kernel(h_hbm, src_hbm, dst_hbm, typ_hbm, rel_hbm, out_hbm, rel_v,
                  head, tail, srcr, dstr, typr, scr, tp_v, msem, hsem, tsem,
                  wsem):
    wid = _worker_id()
    pltpu.sync_copy(rel_hbm, rel_v)
    base = wid * NCH

    def start_meta(c, q):
        eb = (base + c) * C
        pltpu.async_copy(src_hbm.at[pl.ds(eb, C)], srcr[q], msem[q])
        pltpu.async_copy(dst_hbm.at[pl.ds(eb, C)], dstr[q], msem[q])
        pltpu.async_copy(typ_hbm.at[pl.ds(eb, C)], typr[q], msem[q])

    def wait_meta(c, q):
        eb = (base + c) * C
        pltpu.make_async_copy(src_hbm.at[pl.ds(eb, C)], srcr[q], msem[q]).wait()
        pltpu.make_async_copy(dst_hbm.at[pl.ds(eb, C)], dstr[q], msem[q]).wait()
        pltpu.make_async_copy(typ_hbm.at[pl.ds(eb, C)], typr[q], msem[q]).wait()

    def start_gathers(q, k):
        pltpu.async_copy(h_hbm.at[srcr[q]], head[k], hsem[k])
        pltpu.async_copy(h_hbm.at[dstr[q]], tail[k], tsem[k])

    def wait_gathers(q, k):
        pltpu.make_async_copy(h_hbm.at[srcr[q]], head[k], hsem[k]).wait()
        pltpu.make_async_copy(h_hbm.at[dstr[q]], tail[k], tsem[k]).wait()

    def start_out(c, k):
        eb = (base + c) * C
        pltpu.async_copy(scr[k], out_hbm.at[pl.ds(eb, C)], wsem[k])

    def wait_out(c, k):
        eb = (base + c) * C
        pltpu.make_async_copy(scr[k], out_hbm.at[pl.ds(eb, C)], wsem[k]).wait()

    lane = lax.iota(i32, L)

    def compute(q, k):
        def grp(g, _):
            tv16 = typr[q][pl.ds(g * L, L)]
            for j in range(L):
                e = g * L + j
                te = tv16[j]
                acc = jnp.zeros((L,), f32)
                for d in range(D // L):
                    sl = pl.ds(d * L, L)
                    acc = acc + head[k][e, sl] * rel_v[te, sl] * tail[k][e, sl]
                plsc.store_scatter(tp_v, [lane * L + j], acc)
            res = jnp.zeros((L,), f32)
            for l in range(L):
                res = res + tp_v[pl.ds(l * L, L)]
            scr[k][pl.ds(g * L, L)] = res
            return 0

        lax.fori_loop(0, C // L, grp, 0)

    for c in range(3):
        start_meta(c, c)
    wait_meta(0, 0)
    start_gathers(0, 0)

    def slot(i, _):
        for j in range(4):
            c = 4 * i + j
            k = j % 2
            kn = (j + 1) % 2
            q = j
            qn = (j + 1) % 4
            qp = (j + 3) % 4

            @pl.when(c + 1 < NCH)
            def _():
                wait_meta(c + 1, qn)
                start_gathers(qn, kn)

                @pl.when(c + 3 < NCH)
                def _():
                    start_meta(c + 3, qp)

            @pl.when(c < NCH)
            def _():
                wait_gathers(q, k)

                @pl.when(c >= 2)
                def _():
                    wait_out(c - 2, k)

                compute(q, k)
                start_out(c, k)
        return 0

    lax.fori_loop(0, NCH // 4, slot, 0)
    for c in (NCH - 2, NCH - 1):
        wait_out(c, c % 2)


BN = 1000


def _mm_body(x_ref, w_ref, o_ref):
    o_ref[0] = jnp.dot(x_ref[...], w_ref[0], preferred_element_type=f32,
                       precision=lax.Precision.HIGHEST)


_mm = pl.pallas_call(
    _mm_body,
    grid=(N // BN, R + 1),
    in_specs=[
        pl.BlockSpec((BN, D), lambda nb, r: (nb, 0)),
        pl.BlockSpec((1, D, D), lambda nb, r: (r, 0, 0)),
    ],
    out_specs=pl.BlockSpec((1, BN, D), lambda nb, r: (r, nb, 0)),
    out_shape=jax.ShapeDtypeStruct((R + 1, N, D), f32),
)


def _comb_body(p_ref, y_ref, b_ref, o_ref):
    o_ref[...] = jnp.maximum(p_ref[0] + p_ref[1] + y_ref[0] + b_ref[...], 0.0)


_comb = pl.pallas_call(
    _comb_body,
    grid=(N // BN,),
    in_specs=[
        pl.BlockSpec((2, BN, D), lambda nb: (0, nb, 0)),
        pl.BlockSpec((1, BN, D), lambda nb: (R, nb, 0)),
        pl.BlockSpec((1, D), lambda nb: (0, 0)),
    ],
    out_specs=pl.BlockSpec((BN, D), lambda nb: (nb, 0)),
    out_shape=jax.ShapeDtypeStruct((N, D), f32),
)


def kernel(x, edge_index, edge_type, W1, root1, b1, W2, root2, b2, rel_emb):
    pad = EP - E
    spread = jnp.arange(pad, dtype=i32) % N
    srcp = jnp.concatenate([edge_index[0], spread])
    dstp = jnp.concatenate([edge_index[1], spread])
    typp = jnp.concatenate([edge_type, jnp.arange(pad, dtype=i32) % R])
    dst_cnt = jnp.concatenate([edge_index[1], jnp.full((pad,), N, i32)])

    norm, gidx = _norm_kernel(dst_cnt, typp, srcp)

    W1a = jnp.concatenate([W1, root1[None]], axis=0)
    y1 = _mm(x, W1a)
    p1 = _layer_kernel(y1.reshape((R + 1) * N, D), gidx, dstp, norm)
    h1 = _comb(p1, y1, b1.reshape(1, D))

    W2a = jnp.concatenate([W2, root2[None]], axis=0)
    y2 = _mm(h1, W2a)
    p2 = _layer_kernel(y2.reshape((R + 1) * N, D), gidx, dstp, norm)
    h2 = _comb(p2, y2, b2.reshape(1, D))

    return _score_kernel(h2, srcp, dstp, typp, rel_emb)[:E]


# speedup vs baseline: 33.4925x; 1.0002x over previous
import functools

import jax
import jax.numpy as jnp
from jax import lax
from jax.experimental import pallas as pl
from jax.experimental.pallas import tpu as pltpu
from jax.experimental.pallas import tpu_sc as plsc

N = 10000
R = 16
D = 128
E = 320000

NC = 2
NS = 16
L = 16
NW = NC * NS
C = 128
EP = 327680
NCH = EP // NW // C
NCC = EP // NS // C
NR2 = 161280
ZS = NR2 // NS
RU = 80
NU = N // RU

f32 = jnp.float32
i32 = jnp.int32

_mesh = plsc.VectorSubcoreMesh(core_axis_name="c", subcore_axis_name="s")
_params = pltpu.CompilerParams(needs_layout_passes=False)


def _worker_id():
    return lax.axis_index("s") * NC + lax.axis_index("c")


@functools.partial(
    pl.kernel,
    out_type=[
        jax.ShapeDtypeStruct((EP,), f32),
        jax.ShapeDtypeStruct((EP,), i32),
    ],
    mesh=_mesh,
    compiler_params=_params,
    scratch_types=[
        pltpu.VMEM_SHARED((NR2,), f32),
        pltpu.VMEM((2048,), f32),
        [pltpu.VMEM((C,), i32)] * 4,
        [pltpu.VMEM((C,), i32)] * 4,
        [pltpu.VMEM((C,), i32)] * 4,
        [pltpu.VMEM((C,), i32)] * 4,
        [pltpu.VMEM((C,), f32)] * 4,
        [pltpu.VMEM((C,), f32)] * 2,
        [pltpu.VMEM((C,), i32)] * 2,
        [pltpu.SemaphoreType.DMA] * 4,
        [pltpu.SemaphoreType.DMA] * 4,
        [pltpu.SemaphoreType.DMA] * 2,
    ],
)
def _norm_kernel(dst_hbm, typ_hbm, src_hbm, norm_hbm, gidx_hbm, counts_sh,
                 stage_v, dstr, typr, srcr, combr, cntr, nor, gor, msem, csem,
                 wsem):
    sid = lax.axis_index("s")
    wid = _worker_id()

    def fill(i, _):
        stage_v[pl.ds(i * L, L)] = jnp.zeros((L,), f32)
        return 0

    lax.fori_loop(0, 2048 // L, fill, 0)

    def zero_counts(j, _):
        pltpu.sync_copy(stage_v.at[pl.ds(0, 2016)],
                        counts_sh.at[pl.ds(sid * ZS + j * 2016, 2016)])
        return 0

    lax.fori_loop(0, ZS // 2016, zero_counts, 0)
    plsc.subcore_barrier()

    for g in range(C // L):
        stage_v[pl.ds(g * L, L)] = jnp.ones((L,), f32)
    ones = stage_v.at[pl.ds(0, C)]

    cbase = sid * NCC

    def start_meta2(c, q):
        eb = (cbase + c) * C
        pltpu.async_copy(dst_hbm.at[pl.ds(eb, C)], dstr[q], msem[q])
        pltpu.async_copy(typ_hbm.at[pl.ds(eb, C)], typr[q], msem[q])

    def wait_meta2(c, q):
        eb = (cbase + c) * C
        pltpu.make_async_copy(dst_hbm.at[pl.ds(eb, C)], dstr[q], msem[q]).wait()
        pltpu.make_async_copy(typ_hbm.at[pl.ds(eb, C)], typr[q], msem[q]).wait()

    def comb_compute(q):
        for g in range(C // L):
            gl = pl.ds(g * L, L)
            combr[q][gl] = dstr[q][gl] * R + typr[q][gl]

    def start_cscatter(q):
        pltpu.async_copy(ones, counts_sh.at[combr[q]], csem[q], add=True)

    def wait_cscatter(q):
        pltpu.make_async_copy(ones, counts_sh.at[combr[q]], csem[q]).wait()

    for c in range(4):
        start_meta2(c, c)
    for c in range(2):
        wait_meta2(c, c)
        comb_compute(c)
        start_cscatter(c)
    start_meta2(4, 0)
    start_meta2(5, 1)

    def count_slot(i, _):
        for j in range(4):
            c = 4 * i + j
            q2 = (j + 2) % 4

            @pl.when(c + 2 < NCC)
            def _():
                @pl.when(c >= 2)
                def _():
                    wait_cscatter(q2)

                wait_meta2(c + 2, q2)
                comb_compute(q2)
                start_cscatter(q2)

                @pl.when(c + 6 < NCC)
                def _():
                    start_meta2(c + 6, q2)
        return 0

    lax.fori_loop(0, NCC // 4, count_slot, 0)
    for q in range(4):
        wait_cscatter(q)
    plsc.subcore_barrier()

    base = wid * NCH

    def start_meta3(c, q):
        eb = (base + c) * C
        pltpu.async_copy(dst_hbm.at[pl.ds(eb, C)], dstr[q], msem[q])
        pltpu.async_copy(typ_hbm.at[pl.ds(eb, C)], typr[q], msem[q])
        pltpu.async_copy(src_hbm.at[pl.ds(eb, C)], srcr[q], msem[q])

    def wait_meta3(c, q):
        eb = (base + c) * C
        pltpu.make_async_copy(dst_hbm.at[pl.ds(eb, C)], dstr[q], msem[q]).wait()
        pltpu.make_async_copy(typ_hbm.at[pl.ds(eb, C)], typr[q], msem[q]).wait()
        pltpu.make_async_copy(src_hbm.at[pl.ds(eb, C)], srcr[q], msem[q]).wait()

    def start_cgather(q):
        pltpu.async_copy(counts_sh.at[combr[q]], cntr[q], csem[q])

    def wait_cgather(q):
        pltpu.make_async_copy(counts_sh.at[combr[q]], cntr[q], csem[q]).wait()

    def start_out(c, k):
        eb = (base + c) * C
        pltpu.async_copy(nor[k], norm_hbm.at[pl.ds(eb, C)], wsem[k])
        pltpu.async_copy(gor[k], gidx_hbm.at[pl.ds(eb, C)], wsem[k])

    def wait_out(c, k):
        eb = (base + c) * C
        pltpu.make_async_copy(nor[k], norm_hbm.at[pl.ds(eb, C)], wsem[k]).wait()
        pltpu.make_async_copy(gor[k], gidx_hbm.at[pl.ds(eb, C)], wsem[k]).wait()

    for c in range(4):
        start_meta3(c, c)
    for c in range(2):
        wait_meta3(c, c)
        comb_compute(c)
        start_cgather(c)

    def norm_slot(i, _):
        for j in range(4):
            c = 4 * i + j
            q = j
            q2 = (j + 2) % 4
            k = j % 2

            @pl.when(c + 2 < NCH)
            def _():
                wait_meta3(c + 2, q2)
                comb_compute(q2)
                start_cgather(q2)

            wait_cgather(q)

            @pl.when(c >= 2)
            def _():
                wait_out(c - 2, k)

            for g in range(C // L):
                gl = pl.ds(g * L, L)
                nv = 1.0 / jnp.maximum(cntr[q][gl], 1.0)
                nor[k][gl] = jnp.where(dstr[q][gl] == N, 0.0, nv)
                gor[k][gl] = typr[q][gl] * N + srcr[q][gl]
            start_out(c, k)

            @pl.when(c + 4 < NCH)
            def _():
                start_meta3(c + 4, q)
        return 0

    lax.fori_loop(0, NCH // 4, norm_slot, 0)
    for c in (NCH - 2, NCH - 1):
        wait_out(c, c % 2)


@functools.partial(
    pl.kernel,
    out_type=jax.ShapeDtypeStruct((NC, N, D), f32),
    mesh=_mesh,
    compiler_params=_params,
    scratch_types=[
        pltpu.VMEM_SHARED((N + 8, D), f32),
        [pltpu.VMEM((C, D), f32)] * 2,
        [pltpu.VMEM((C,), i32)] * 4,
        [pltpu.VMEM((C,), i32)] * 4,
        [pltpu.VMEM((C,), f32)] * 4,
        [pltpu.SemaphoreType.DMA] * 4,
        [pltpu.SemaphoreType.DMA] * 2,
        [pltpu.SemaphoreType.DMA] * 2,
    ],
)
def _layer_kernel(y_hbm, gidx_hbm, dst_hbm, norm_hbm, out_hbm, agg_sh,
                  rows, gr, dr, nr, msem, gsem, ssem):
    cid = lax.axis_index("c")
    sid = lax.axis_index("s")
    wid = _worker_id()

    def fill(i, _):
        for d in range(D // L):
            rows[0][i, pl.ds(d * L, L)] = jnp.zeros((L,), f32)
        return 0

    lax.fori_loop(0, RU, fill, 0)

    n_units = jnp.where(sid < NU - (NU // NS) * NS, NU // NS + 1, NU // NS)

    def zero_agg(u, _):
        pltpu.sync_copy(rows[0].at[pl.ds(0, RU)],
                        agg_sh.at[pl.ds((sid + u * NS) * RU, RU)])
        return 0

    lax.fori_loop(0, n_units, zero_agg, 0)
    plsc.subcore_barrier()

    base = wid * NCH

    def start_meta(c, q):
        eb = (base + c) * C
        pltpu.async_copy(gidx_hbm.at[pl.ds(eb, C)], gr[q], msem[q])
        pltpu.async_copy(dst_hbm.at[pl.ds(eb, C)], dr[q], msem[q])
        pltpu.async_copy(norm_hbm.at[pl.ds(eb, C)], nr[q], msem[q])

    def wait_meta(c, q):
        eb = (base + c) * C
        pltpu.make_async_copy(gidx_hbm.at[pl.ds(eb, C)], gr[q], msem[q]).wait()
        pltpu.make_async_copy(dst_hbm.at[pl.ds(eb, C)], dr[q], msem[q]).wait()
        pltpu.make_async_copy(norm_hbm.at[pl.ds(eb, C)], nr[q], msem[q]).wait()

    def start_gather(q, k):
        pltpu.async_copy(y_hbm.at[gr[q]], rows[k], gsem[k])

    def wait_gather(q, k):
        pltpu.make_async_copy(y_hbm.at[gr[q]], rows[k], gsem[k]).wait()

    def start_scatter(q, k):
        pltpu.async_copy(rows[k], agg_sh.at[dr[q]], ssem[k], add=True)

    def wait_scatter(q, k):
        pltpu.make_async_copy(rows[k], agg_sh.at[dr[q]], ssem[k]).wait()

    def scale(q, k):
        def body(g, _):
            nv16 = nr[q][pl.ds(g * L, L)]
            for j in range(L):
                bv = jnp.full((L,), nv16[j], f32)
                for d in range(D // L):
                    sl = pl.ds(d * L, L)
                    rows[k][g * L + j, sl] = rows[k][g * L + j, sl] * bv
            return 0

        lax.fori_loop(0, C // L, body, 0)

    for c in range(3):
        start_meta(c, c)
    wait_meta(0, 0)
    start_gather(0, 0)

    def slot(i, _):
        for j in range(4):
            c = 4 * i + j
            k = j % 2
            kn = (j + 1) % 2
            q = j
            qn = (j + 1) % 4
            qp = (j + 3) % 4

            @pl.when(c + 1 < NCH)
            def _():
                @pl.when(c >= 1)
                def _():
                    wait_scatter(qp, kn)

                wait_meta(c + 1, qn)
                start_gather(qn, kn)

                @pl.when(c + 3 < NCH)
                def _():
                    start_meta(c + 3, qp)

            @pl.when(c < NCH)
            def _():
                wait_gather(q, k)
                scale(q, k)
                start_scatter(q, k)
        return 0

    lax.fori_loop(0, NCH // 4, slot, 0)
    wait_scatter((NCH - 2) % 4, (NCH - 2) % 2)
    wait_scatter((NCH - 1) % 4, (NCH - 1) % 2)
    plsc.subcore_barrier()

    def writeback(u, _):
        rbase = (sid + u * NS) * RU
        pltpu.sync_copy(agg_sh.at[pl.ds(rbase, RU)], rows[0].at[pl.ds(0, RU)])
        pltpu.sync_copy(rows[0].at[pl.ds(0, RU)],
                        out_hbm.at[cid, pl.ds(rbase, RU)])
        return 0

    lax.fori_loop(0, n_units, writeback, 0)


@functools.partial(
    pl.kernel,
    out_type=jax.ShapeDtypeStruct((EP,), f32),
    mesh=_mesh,
    compiler_params=_params,
    scratch_types=[
        pltpu.VMEM((R, D), f32),
        [pltpu.VMEM((C, D), f32)] * 2,
        [pltpu.VMEM((C, D), f32)] * 2,
        [pltpu.VMEM((C,), i32)] * 4,
        [pltpu.VMEM((C,), i32)] * 4,
        [pltpu.VMEM((C,), i32)] * 4,
        [pltpu.VMEM((C,), f32)] * 2,
        pltpu.VMEM((L * L,), f32),
        [pltpu.SemaphoreType.DMA] * 4,
        [pltpu.SemaphoreType.DMA] * 2,
        [pltpu.SemaphoreType.DMA] * 2,
        [pltpu.SemaphoreType.DMA] * 2,
    ],
)
def _score_kernel(h_hbm, src_hbm, dst_hbm, typ_hbm, rel_hbm, out_hbm, rel_v,
                  head, tail, srcr, dstr, typr, scr, tp_v, msem, hsem, tsem,
                  wsem):
    wid = _worker_id()
    pltpu.sync_copy(rel_hbm, rel_v)
    base = wid * NCH

    def start_meta(c, q):
        eb = (base + c) * C
        pltpu.async_copy(src_hbm.at[pl.ds(eb, C)], srcr[q], msem[q])
        pltpu.async_copy(dst_hbm.at[pl.ds(eb, C)], dstr[q], msem[q])
        pltpu.async_copy(typ_hbm.at[pl.ds(eb, C)], typr[q], msem[q])

    def wait_meta(c, q):
        eb = (base + c) * C
        pltpu.make_async_copy(src_hbm.at[pl.ds(eb, C)], srcr[q], msem[q]).wait()
        pltpu.make_async_copy(dst_hbm.at[pl.ds(eb, C)], dstr[q], msem[q]).wait()
        pltpu.make_async_copy(typ_hbm.at[pl.ds(eb, C)], typr[q], msem[q]).wait()

    def start_gathers(q, k):
        pltpu.async_copy(h_hbm.at[srcr[q]], head[k], hsem[k])
        pltpu.async_copy(h_hbm.at[dstr[q]], tail[k], tsem[k])

    def wait_gathers(q, k):
        pltpu.make_async_copy(h_hbm.at[srcr[q]], head[k], hsem[k]).wait()
        pltpu.make_async_copy(h_hbm.at[dstr[q]], tail[k], tsem[k]).wait()

    def start_out(c, k):
        eb = (base + c) * C
        pltpu.async_copy(scr[k], out_hbm.at[pl.ds(eb, C)], wsem[k])

    def wait_out(c, k):
        eb = (base + c) * C
        pltpu.make_async_copy(scr[k], out_hbm.at[pl.ds(eb, C)], wsem[k]).wait()

    lane = lax.iota(i32, L)

    def compute(q, k):
        def grp(g, _):
            tv16 = typr[q][pl.ds(g * L, L)]
            for j in range(L):
                e = g * L + j
                te = tv16[j]
                acc = jnp.zeros((L,), f32)
                for d in range(D // L):
                    sl = pl.ds(d * L, L)
                    acc = acc + head[k][e, sl] * rel_v[te, sl] * tail[k][e, sl]
                plsc.store_scatter(tp_v, [lane * L + j], acc)
            res = jnp.zeros((L,), f32)
            for l in range(L):
                res = res + tp_v[pl.ds(l * L, L)]
            scr[k][pl.ds(g * L, L)] = res
            return 0

        lax.fori_loop(0, C // L, grp, 0)

    for c in range(3):
        start_meta(c, c)
    wait_meta(0, 0)
    start_gathers(0, 0)

    def slot(i, _):
        for j in range(4):
            c = 4 * i + j
            k = j % 2
            kn = (j + 1) % 2
            q = j
            qn = (j + 1) % 4
            qp = (j + 3) % 4

            @pl.when(c + 1 < NCH)
            def _():
                wait_meta(c + 1, qn)
                start_gathers(qn, kn)

                @pl.when(c + 3 < NCH)
                def _():
                    start_meta(c + 3, qp)

            @pl.when(c < NCH)
            def _():
                wait_gathers(q, k)

                @pl.when(c >= 2)
                def _():
                    wait_out(c - 2, k)

                compute(q, k)
                start_out(c, k)
        return 0

    lax.fori_loop(0, NCH // 4, slot, 0)
    for c in (NCH - 2, NCH - 1):
        wait_out(c, c % 2)


BN = 1000


def _mm_body(x_ref, w_ref, o_ref):
    o_ref[0] = jnp.dot(x_ref[...], w_ref[0], preferred_element_type=f32,
                       precision=lax.Precision.HIGHEST)


_mm = pl.pallas_call(
    _mm_body,
    grid=(N // BN, R + 1),
    in_specs=[
        pl.BlockSpec((BN, D), lambda nb, r: (nb, 0)),
        pl.BlockSpec((1, D, D), lambda nb, r: (r, 0, 0)),
    ],
    out_specs=pl.BlockSpec((1, BN, D), lambda nb, r: (r, nb, 0)),
    out_shape=jax.ShapeDtypeStruct((R + 1, N, D), f32),
)


def _mmc_body(p_ref, y_ref, b_ref, w_ref, o_ref, h_ref):
    @pl.when(pl.program_id(1) == 0)
    def _():
        h_ref[...] = jnp.maximum(p_ref[0] + p_ref[1] + y_ref[0] + b_ref[...],
                                 0.0)

    o_ref[0] = jnp.dot(h_ref[...], w_ref[0], preferred_element_type=f32,
                       precision=lax.Precision.HIGHEST)


_mmc = pl.pallas_call(
    _mmc_body,
    grid=(N // BN, R + 1),
    in_specs=[
        pl.BlockSpec((2, BN, D), lambda nb, r: (0, nb, 0)),
        pl.BlockSpec((1, BN, D), lambda nb, r: (R, nb, 0)),
        pl.BlockSpec((1, D), lambda nb, r: (0, 0)),
        pl.BlockSpec((1, D, D), lambda nb, r: (r, 0, 0)),
    ],
    out_specs=pl.BlockSpec((1, BN, D), lambda nb, r: (r, nb, 0)),
    out_shape=jax.ShapeDtypeStruct((R + 1, N, D), f32),
    scratch_shapes=[pltpu.VMEM((BN, D), f32)],
)


def _comb_body(p_ref, y_ref, b_ref, o_ref):
    o_ref[...] = jnp.maximum(p_ref[0] + p_ref[1] + y_ref[0] + b_ref[...], 0.0)


_comb = pl.pallas_call(
    _comb_body,
    grid=(N // BN,),
    in_specs=[
        pl.BlockSpec((2, BN, D), lambda nb: (0, nb, 0)),
        pl.BlockSpec((1, BN, D), lambda nb: (R, nb, 0)),
        pl.BlockSpec((1, D), lambda nb: (0, 0)),
    ],
    out_specs=pl.BlockSpec((BN, D), lambda nb: (nb, 0)),
    out_shape=jax.ShapeDtypeStruct((N, D), f32),
)


def kernel(x, edge_index, edge_type, W1, root1, b1, W2, root2, b2, rel_emb):
    pad = EP - E
    spread = jnp.arange(pad, dtype=i32) % N
    srcp = jnp.concatenate([edge_index[0], spread])
    dstp = jnp.concatenate([edge_index[1], spread])
    typp = jnp.concatenate([edge_type, jnp.arange(pad, dtype=i32) % R])
    dst_cnt = jnp.concatenate([edge_index[1], jnp.full((pad,), N, i32)])

    norm, gidx = _norm_kernel(dst_cnt, typp, srcp)

    W1a = jnp.concatenate([W1, root1[None]], axis=0)
    y1 = _mm(x, W1a)
    p1 = _layer_kernel(y1.reshape((R + 1) * N, D), gidx, dstp, norm)

    W2a = jnp.concatenate([W2, root2[None]], axis=0)
    y2 = _mmc(p1, y1, b1.reshape(1, D), W2a)
    p2 = _layer_kernel(y2.reshape((R + 1) * N, D), gidx, dstp, norm)
    h2 = _comb(p2, y2, b2.reshape(1, D))

    return _score_kernel(h2, srcp, dstp, typp, rel_emb)[:E]


# speedup vs baseline: 35.1556x; 1.0497x over previous
import functools

import jax
import jax.numpy as jnp
from jax import lax
from jax.experimental import pallas as pl
from jax.experimental.pallas import tpu as pltpu
from jax.experimental.pallas import tpu_sc as plsc

N = 10000
R = 16
D = 128
E = 320000

NC = 2
NS = 16
L = 16
NW = NC * NS
C = 128
EP = 327680
NCH = EP // NW // C
NCC = EP // NS // C
NR2 = 161280
ZS = NR2 // NS
RU = 80
NU = N // RU

f32 = jnp.float32
i32 = jnp.int32

_mesh = plsc.VectorSubcoreMesh(core_axis_name="c", subcore_axis_name="s")
_params = pltpu.CompilerParams(needs_layout_passes=False)


def _worker_id():
    return lax.axis_index("s") * NC + lax.axis_index("c")


@functools.partial(
    pl.kernel,
    out_type=[
        jax.ShapeDtypeStruct((EP,), f32),
        jax.ShapeDtypeStruct((EP,), i32),
    ],
    mesh=_mesh,
    compiler_params=_params,
    scratch_types=[
        pltpu.VMEM_SHARED((NR2,), f32),
        pltpu.VMEM((2048,), f32),
        [pltpu.VMEM((C,), i32)] * 4,
        [pltpu.VMEM((C,), i32)] * 4,
        [pltpu.VMEM((C,), i32)] * 4,
        [pltpu.VMEM((C,), i32)] * 4,
        [pltpu.VMEM((C,), f32)] * 4,
        [pltpu.VMEM((C,), f32)] * 2,
        [pltpu.VMEM((C,), i32)] * 2,
        [pltpu.SemaphoreType.DMA] * 4,
        [pltpu.SemaphoreType.DMA] * 4,
        [pltpu.SemaphoreType.DMA] * 2,
    ],
)
def _norm_kernel(dst_hbm, typ_hbm, src_hbm, norm_hbm, gidx_hbm, counts_sh,
                 stage_v, dstr, typr, srcr, combr, cntr, nor, gor, msem, csem,
                 wsem):
    sid = lax.axis_index("s")
    wid = _worker_id()

    def fill(i, _):
        stage_v[pl.ds(i * L, L)] = jnp.zeros((L,), f32)
        return 0

    lax.fori_loop(0, 2048 // L, fill, 0)

    def zero_counts(j, _):
        pltpu.sync_copy(stage_v.at[pl.ds(0, 2016)],
                        counts_sh.at[pl.ds(sid * ZS + j * 2016, 2016)])
        return 0

    lax.fori_loop(0, ZS // 2016, zero_counts, 0)
    plsc.subcore_barrier()

    for g in range(C // L):
        stage_v[pl.ds(g * L, L)] = jnp.ones((L,), f32)
    ones = stage_v.at[pl.ds(0, C)]

    cbase = sid * NCC

    def start_meta2(c, q):
        eb = (cbase + c) * C
        pltpu.async_copy(dst_hbm.at[pl.ds(eb, C)], dstr[q], msem[q])
        pltpu.async_copy(typ_hbm.at[pl.ds(eb, C)], typr[q], msem[q])

    def wait_meta2(c, q):
        eb = (cbase + c) * C
        pltpu.make_async_copy(dst_hbm.at[pl.ds(eb, C)], dstr[q], msem[q]).wait()
        pltpu.make_async_copy(typ_hbm.at[pl.ds(eb, C)], typr[q], msem[q]).wait()

    def comb_compute(q):
        for g in range(C // L):
            gl = pl.ds(g * L, L)
            combr[q][gl] = dstr[q][gl] * R + typr[q][gl]

    def start_cscatter(q):
        pltpu.async_copy(ones, counts_sh.at[combr[q]], csem[q], add=True)

    def wait_cscatter(q):
        pltpu.make_async_copy(ones, counts_sh.at[combr[q]], csem[q]).wait()

    for c in range(4):
        start_meta2(c, c)
    for c in range(2):
        wait_meta2(c, c)
        comb_compute(c)
        start_cscatter(c)
    start_meta2(4, 0)
    start_meta2(5, 1)

    def count_slot(i, _):
        for j in range(4):
            c = 4 * i + j
            q2 = (j + 2) % 4

            @pl.when(c + 2 < NCC)
            def _():
                @pl.when(c >= 2)
                def _():
                    wait_cscatter(q2)

                wait_meta2(c + 2, q2)
                comb_compute(q2)
                start_cscatter(q2)

                @pl.when(c + 6 < NCC)
                def _():
                    start_meta2(c + 6, q2)
        return 0

    lax.fori_loop(0, NCC // 4, count_slot, 0)
    for q in range(4):
        wait_cscatter(q)
    plsc.subcore_barrier()

    base = wid * NCH

    def start_meta3(c, q):
        eb = (base + c) * C
        pltpu.async_copy(dst_hbm.at[pl.ds(eb, C)], dstr[q], msem[q])
        pltpu.async_copy(typ_hbm.at[pl.ds(eb, C)], typr[q], msem[q])
        pltpu.async_copy(src_hbm.at[pl.ds(eb, C)], srcr[q], msem[q])

    def wait_meta3(c, q):
        eb = (base + c) * C
        pltpu.make_async_copy(dst_hbm.at[pl.ds(eb, C)], dstr[q], msem[q]).wait()
        pltpu.make_async_copy(typ_hbm.at[pl.ds(eb, C)], typr[q], msem[q]).wait()
        pltpu.make_async_copy(src_hbm.at[pl.ds(eb, C)], srcr[q], msem[q]).wait()

    def start_cgather(q):
        pltpu.async_copy(counts_sh.at[combr[q]], cntr[q], csem[q])

    def wait_cgather(q):
        pltpu.make_async_copy(counts_sh.at[combr[q]], cntr[q], csem[q]).wait()

    def start_out(c, k):
        eb = (base + c) * C
        pltpu.async_copy(nor[k], norm_hbm.at[pl.ds(eb, C)], wsem[k])
        pltpu.async_copy(gor[k], gidx_hbm.at[pl.ds(eb, C)], wsem[k])

    def wait_out(c, k):
        eb = (base + c) * C
        pltpu.make_async_copy(nor[k], norm_hbm.at[pl.ds(eb, C)], wsem[k]).wait()
        pltpu.make_async_copy(gor[k], gidx_hbm.at[pl.ds(eb, C)], wsem[k]).wait()

    for c in range(4):
        start_meta3(c, c)
    for c in range(2):
        wait_meta3(c, c)
        comb_compute(c)
        start_cgather(c)

    def norm_slot(i, _):
        for j in range(4):
            c = 4 * i + j
            q = j
            q2 = (j + 2) % 4
            k = j % 2

            @pl.when(c + 2 < NCH)
            def _():
                wait_meta3(c + 2, q2)
                comb_compute(q2)
                start_cgather(q2)

            wait_cgather(q)

            @pl.when(c >= 2)
            def _():
                wait_out(c - 2, k)

            for g in range(C // L):
                gl = pl.ds(g * L, L)
                nv = 1.0 / jnp.maximum(cntr[q][gl], 1.0)
                nor[k][gl] = jnp.where(dstr[q][gl] == N, 0.0, nv)
                gor[k][gl] = srcr[q][gl] * (R + 1) + typr[q][gl]
            start_out(c, k)

            @pl.when(c + 4 < NCH)
            def _():
                start_meta3(c + 4, q)
        return 0

    lax.fori_loop(0, NCH // 4, norm_slot, 0)
    for c in (NCH - 2, NCH - 1):
        wait_out(c, c % 2)


@functools.partial(
    pl.kernel,
    out_type=jax.ShapeDtypeStruct((NC, N, D), f32),
    mesh=_mesh,
    compiler_params=_params,
    scratch_types=[
        pltpu.VMEM_SHARED((N + 8, D), f32),
        [pltpu.VMEM((C, D), f32)] * 2,
        [pltpu.VMEM((C,), i32)] * 4,
        [pltpu.VMEM((C,), i32)] * 4,
        [pltpu.VMEM((C,), f32)] * 4,
        [pltpu.SemaphoreType.DMA] * 4,
        [pltpu.SemaphoreType.DMA] * 2,
        [pltpu.SemaphoreType.DMA] * 2,
    ],
)
def _layer_kernel(y_hbm, gidx_hbm, dst_hbm, norm_hbm, out_hbm, agg_sh,
                  rows, gr, dr, nr, msem, gsem, ssem):
    cid = lax.axis_index("c")
    sid = lax.axis_index("s")
    wid = _worker_id()

    def fill(i, _):
        for d in range(D // L):
            rows[0][i, pl.ds(d * L, L)] = jnp.zeros((L,), f32)
        return 0

    lax.fori_loop(0, RU, fill, 0)

    n_units = jnp.where(sid < NU - (NU // NS) * NS, NU // NS + 1, NU // NS)

    def zero_agg(u, _):
        pltpu.sync_copy(rows[0].at[pl.ds(0, RU)],
                        agg_sh.at[pl.ds((sid + u * NS) * RU, RU)])
        return 0

    lax.fori_loop(0, n_units, zero_agg, 0)
    plsc.subcore_barrier()

    base = wid * NCH

    def start_meta(c, q):
        eb = (base + c) * C
        pltpu.async_copy(gidx_hbm.at[pl.ds(eb, C)], gr[q], msem[q])
        pltpu.async_copy(dst_hbm.at[pl.ds(eb, C)], dr[q], msem[q])
        pltpu.async_copy(norm_hbm.at[pl.ds(eb, C)], nr[q], msem[q])

    def wait_meta(c, q):
        eb = (base + c) * C
        pltpu.make_async_copy(gidx_hbm.at[pl.ds(eb, C)], gr[q], msem[q]).wait()
        pltpu.make_async_copy(dst_hbm.at[pl.ds(eb, C)], dr[q], msem[q]).wait()
        pltpu.make_async_copy(norm_hbm.at[pl.ds(eb, C)], nr[q], msem[q]).wait()

    def start_gather(q, k):
        pltpu.async_copy(y_hbm.at[gr[q]], rows[k], gsem[k])

    def wait_gather(q, k):
        pltpu.make_async_copy(y_hbm.at[gr[q]], rows[k], gsem[k]).wait()

    def start_scatter(q, k):
        pltpu.async_copy(rows[k], agg_sh.at[dr[q]], ssem[k], add=True)

    def wait_scatter(q, k):
        pltpu.make_async_copy(rows[k], agg_sh.at[dr[q]], ssem[k]).wait()

    def scale(q, k):
        def body(g, _):
            nv16 = nr[q][pl.ds(g * L, L)]
            for j in range(L):
                bv = jnp.full((L,), nv16[j], f32)
                for d in range(D // L):
                    sl = pl.ds(d * L, L)
                    rows[k][g * L + j, sl] = rows[k][g * L + j, sl] * bv
            return 0

        lax.fori_loop(0, C // L, body, 0)

    for c in range(3):
        start_meta(c, c)
    wait_meta(0, 0)
    start_gather(0, 0)

    def slot(i, _):
        for j in range(4):
            c = 4 * i + j
            k = j % 2
            kn = (j + 1) % 2
            q = j
            qn = (j + 1) % 4
            qp = (j + 3) % 4

            @pl.when(c + 1 < NCH)
            def _():
                @pl.when(c >= 1)
                def _():
                    wait_scatter(qp, kn)

                wait_meta(c + 1, qn)
                start_gather(qn, kn)

                @pl.when(c + 3 < NCH)
                def _():
                    start_meta(c + 3, qp)

            @pl.when(c < NCH)
            def _():
                wait_gather(q, k)
                scale(q, k)
                start_scatter(q, k)
        return 0

    lax.fori_loop(0, NCH // 4, slot, 0)
    wait_scatter((NCH - 2) % 4, (NCH - 2) % 2)
    wait_scatter((NCH - 1) % 4, (NCH - 1) % 2)
    plsc.subcore_barrier()

    def writeback(u, _):
        rbase = (sid + u * NS) * RU
        pltpu.sync_copy(agg_sh.at[pl.ds(rbase, RU)], rows[0].at[pl.ds(0, RU)])
        pltpu.sync_copy(rows[0].at[pl.ds(0, RU)],
                        out_hbm.at[cid, pl.ds(rbase, RU)])
        return 0

    lax.fori_loop(0, n_units, writeback, 0)


@functools.partial(
    pl.kernel,
    out_type=jax.ShapeDtypeStruct((EP,), f32),
    mesh=_mesh,
    compiler_params=_params,
    scratch_types=[
        pltpu.VMEM((R, D), f32),
        [pltpu.VMEM((C, D), f32)] * 2,
        [pltpu.VMEM((C, D), f32)] * 2,
        [pltpu.VMEM((C,), i32)] * 4,
        [pltpu.VMEM((C,), i32)] * 4,
        [pltpu.VMEM((C,), i32)] * 4,
        [pltpu.VMEM((C,), f32)] * 2,
        pltpu.VMEM((L * L,), f32),
        [pltpu.SemaphoreType.DMA] * 4,
        [pltpu.SemaphoreType.DMA] * 2,
        [pltpu.SemaphoreType.DMA] * 2,
        [pltpu.SemaphoreType.DMA] * 2,
    ],
)
def _score_kernel(h_hbm, src_hbm, dst_hbm, typ_hbm, rel_hbm, out_hbm, rel_v,
                  head, tail, srcr, dstr, typr, scr, tp_v, msem, hsem, tsem,
                  wsem):
    wid = _worker_id()
    pltpu.sync_copy(rel_hbm, rel_v)
    base = wid * NCH

    def start_meta(c, q):
        eb = (base + c) * C
        pltpu.async_copy(src_hbm.at[pl.ds(eb, C)], srcr[q], msem[q])
        pltpu.async_copy(dst_hbm.at[pl.ds(eb, C)], dstr[q], msem[q])
        pltpu.async_copy(typ_hbm.at[pl.ds(eb, C)], typr[q], msem[q])

    def wait_meta(c, q):
        eb = (base + c) * C
        pltpu.make_async_copy(src_hbm.at[pl.ds(eb, C)], srcr[q], msem[q]).wait()
        pltpu.make_async_copy(dst_hbm.at[pl.ds(eb, C)], dstr[q], msem[q]).wait()
        pltpu.make_async_copy(typ_hbm.at[pl.ds(eb, C)], typr[q], msem[q]).wait()

    def start_gathers(q, k):
        pltpu.async_copy(h_hbm.at[srcr[q]], head[k], hsem[k])
        pltpu.async_copy(h_hbm.at[dstr[q]], tail[k], tsem[k])

    def wait_gathers(q, k):
        pltpu.make_async_copy(h_hbm.at[srcr[q]], head[k], hsem[k]).wait()
        pltpu.make_async_copy(h_hbm.at[dstr[q]], tail[k], tsem[k]).wait()

    def start_out(c, k):
        eb = (base + c) * C
        pltpu.async_copy(scr[k], out_hbm.at[pl.ds(eb, C)], wsem[k])

    def wait_out(c, k):
        eb = (base + c) * C
        pltpu.make_async_copy(scr[k], out_hbm.at[pl.ds(eb, C)], wsem[k]).wait()

    lane = lax.iota(i32, L)

    def compute(q, k):
        def grp(g, _):
            tv16 = typr[q][pl.ds(g * L, L)]
            for j in range(L):
                e = g * L + j
                te = tv16[j]
                acc = jnp.zeros((L,), f32)
                for d in range(D // L):
                    sl = pl.ds(d * L, L)
                    acc = acc + head[k][e, sl] * rel_v[te, sl] * tail[k][e, sl]
                plsc.store_scatter(tp_v, [lane * L + j], acc)
            res = jnp.zeros((L,), f32)
            for l in range(L):
                res = res + tp_v[pl.ds(l * L, L)]
            scr[k][pl.ds(g * L, L)] = res
            return 0

        lax.fori_loop(0, C // L, grp, 0)

    for c in range(3):
        start_meta(c, c)
    wait_meta(0, 0)
    start_gathers(0, 0)

    def slot(i, _):
        for j in range(4):
            c = 4 * i + j
            k = j % 2
            kn = (j + 1) % 2
            q = j
            qn = (j + 1) % 4
            qp = (j + 3) % 4

            @pl.when(c + 1 < NCH)
            def _():
                wait_meta(c + 1, qn)
                start_gathers(qn, kn)

                @pl.when(c + 3 < NCH)
                def _():
                    start_meta(c + 3, qp)

            @pl.when(c < NCH)
            def _():
                wait_gathers(q, k)

                @pl.when(c >= 2)
                def _():
                    wait_out(c - 2, k)

                compute(q, k)
                start_out(c, k)
        return 0

    lax.fori_loop(0, NCH // 4, slot, 0)
    for c in (NCH - 2, NCH - 1):
        wait_out(c, c % 2)


BN = 1000
DW = (R + 1) * D


def _mm_body(x_ref, w_ref, o_ref):
    o_ref[...] = jnp.dot(x_ref[...], w_ref[...], preferred_element_type=f32,
                         precision=lax.Precision.HIGHEST)


_mm = pl.pallas_call(
    _mm_body,
    grid=(N // BN,),
    in_specs=[
        pl.BlockSpec((BN, D), lambda nb: (nb, 0)),
        pl.BlockSpec((D, DW), lambda nb: (0, 0)),
    ],
    out_specs=pl.BlockSpec((BN, DW), lambda nb: (nb, 0)),
    out_shape=jax.ShapeDtypeStruct((N, DW), f32),
)


def _mmc_body(p_ref, y_ref, b_ref, w_ref, o_ref):
    h = jnp.maximum(p_ref[0] + p_ref[1] + y_ref[...] + b_ref[...], 0.0)
    o_ref[...] = jnp.dot(h, w_ref[...], preferred_element_type=f32,
                         precision=lax.Precision.HIGHEST)


_mmc = pl.pallas_call(
    _mmc_body,
    grid=(N // BN,),
    in_specs=[
        pl.BlockSpec((2, BN, D), lambda nb: (0, nb, 0)),
        pl.BlockSpec((BN, D), lambda nb: (nb, R)),
        pl.BlockSpec((1, D), lambda nb: (0, 0)),
        pl.BlockSpec((D, DW), lambda nb: (0, 0)),
    ],
    out_specs=pl.BlockSpec((BN, DW), lambda nb: (nb, 0)),
    out_shape=jax.ShapeDtypeStruct((N, DW), f32),
)


def _comb_body(p_ref, y_ref, b_ref, o_ref):
    o_ref[...] = jnp.maximum(p_ref[0] + p_ref[1] + y_ref[...] + b_ref[...], 0.0)


_comb = pl.pallas_call(
    _comb_body,
    grid=(N // BN,),
    in_specs=[
        pl.BlockSpec((2, BN, D), lambda nb: (0, nb, 0)),
        pl.BlockSpec((BN, D), lambda nb: (nb, R)),
        pl.BlockSpec((1, D), lambda nb: (0, 0)),
    ],
    out_specs=pl.BlockSpec((BN, D), lambda nb: (nb, 0)),
    out_shape=jax.ShapeDtypeStruct((N, D), f32),
)


def _wcat(W, root):
    return jnp.concatenate(
        [W.transpose(1, 0, 2).reshape(D, R * D), root], axis=1)


def kernel(x, edge_index, edge_type, W1, root1, b1, W2, root2, b2, rel_emb):
    pad = EP - E
    spread = jnp.arange(pad, dtype=i32) % N
    srcp = jnp.concatenate([edge_index[0], spread])
    dstp = jnp.concatenate([edge_index[1], spread])
    typp = jnp.concatenate([edge_type, jnp.arange(pad, dtype=i32) % R])
    dst_cnt = jnp.concatenate([edge_index[1], jnp.full((pad,), N, i32)])

    norm, gidx = _norm_kernel(dst_cnt, typp, srcp)

    y1 = _mm(x, _wcat(W1, root1))
    p1 = _layer_kernel(y1.reshape(N * (R + 1), D), gidx, dstp, norm)

    y2 = _mmc(p1, y1, b1.reshape(1, D), _wcat(W2, root2))
    p2 = _layer_kernel(y2.reshape(N * (R + 1), D), gidx, dstp, norm)
    h2 = _comb(p2, y2, b2.reshape(1, D))

    return _score_kernel(h2, srcp, dstp, typp, rel_emb)[:E]


# speedup vs baseline: 35.2839x; 1.0036x over previous
import functools

import jax
import jax.numpy as jnp
from jax import lax
from jax.experimental import pallas as pl
from jax.experimental.pallas import tpu as pltpu
from jax.experimental.pallas import tpu_sc as plsc

N = 10000
R = 16
D = 128
E = 320000

NC = 2
NS = 16
L = 16
NW = NC * NS
C = 128
EP = 327680
NCH = EP // NW // C
NCC = EP // NS // C
NR2 = 161280
ZS = NR2 // NS
RU = 80
NU = N // RU

f32 = jnp.float32
i32 = jnp.int32

_mesh = plsc.VectorSubcoreMesh(core_axis_name="c", subcore_axis_name="s")
_params = pltpu.CompilerParams(needs_layout_passes=False)


def _worker_id():
    return lax.axis_index("s") * NC + lax.axis_index("c")


@functools.partial(
    pl.kernel,
    out_type=[
        jax.ShapeDtypeStruct((EP,), f32),
        jax.ShapeDtypeStruct((EP,), i32),
    ],
    mesh=_mesh,
    compiler_params=_params,
    scratch_types=[
        pltpu.VMEM_SHARED((NR2,), f32),
        pltpu.VMEM((2048,), f32),
        [pltpu.VMEM((C,), i32)] * 4,
        [pltpu.VMEM((C,), i32)] * 4,
        [pltpu.VMEM((C,), i32)] * 4,
        [pltpu.VMEM((C,), i32)] * 4,
        [pltpu.VMEM((C,), f32)] * 4,
        [pltpu.VMEM((C,), f32)] * 2,
        [pltpu.VMEM((C,), i32)] * 2,
        [pltpu.SemaphoreType.DMA] * 4,
        [pltpu.SemaphoreType.DMA] * 4,
        [pltpu.SemaphoreType.DMA] * 2,
    ],
)
def _norm_kernel(dst_hbm, typ_hbm, src_hbm, norm_hbm, gidx_hbm, counts_sh,
                 stage_v, dstr, typr, srcr, combr, cntr, nor, gor, msem, csem,
                 wsem):
    sid = lax.axis_index("s")
    wid = _worker_id()

    def fill(i, _):
        stage_v[pl.ds(i * L, L)] = jnp.zeros((L,), f32)
        return 0

    lax.fori_loop(0, 2048 // L, fill, 0)

    def zero_counts(j, _):
        pltpu.sync_copy(stage_v.at[pl.ds(0, 2016)],
                        counts_sh.at[pl.ds(sid * ZS + j * 2016, 2016)])
        return 0

    lax.fori_loop(0, ZS // 2016, zero_counts, 0)
    plsc.subcore_barrier()

    for g in range(C // L):
        stage_v[pl.ds(g * L, L)] = jnp.ones((L,), f32)
    ones = stage_v.at[pl.ds(0, C)]

    cbase = sid * NCC

    def start_meta2(c, q):
        eb = (cbase + c) * C
        pltpu.async_copy(dst_hbm.at[pl.ds(eb, C)], dstr[q], msem[q])
        pltpu.async_copy(typ_hbm.at[pl.ds(eb, C)], typr[q], msem[q])

    def wait_meta2(c, q):
        eb = (cbase + c) * C
        pltpu.make_async_copy(dst_hbm.at[pl.ds(eb, C)], dstr[q], msem[q]).wait()
        pltpu.make_async_copy(typ_hbm.at[pl.ds(eb, C)], typr[q], msem[q]).wait()

    def comb_compute(q):
        for g in range(C // L):
            gl = pl.ds(g * L, L)
            combr[q][gl] = dstr[q][gl] * R + typr[q][gl]

    def start_cscatter(q):
        pltpu.async_copy(ones, counts_sh.at[combr[q]], csem[q], add=True)

    def wait_cscatter(q):
        pltpu.make_async_copy(ones, counts_sh.at[combr[q]], csem[q]).wait()

    for c in range(4):
        start_meta2(c, c)
    for c in range(2):
        wait_meta2(c, c)
        comb_compute(c)
        start_cscatter(c)
    start_meta2(4, 0)
    start_meta2(5, 1)

    def count_slot(i, _):
        for j in range(4):
            c = 4 * i + j
            q2 = (j + 2) % 4

            @pl.when(c + 2 < NCC)
            def _():
                @pl.when(c >= 2)
                def _():
                    wait_cscatter(q2)

                wait_meta2(c + 2, q2)
                comb_compute(q2)
                start_cscatter(q2)

                @pl.when(c + 6 < NCC)
                def _():
                    start_meta2(c + 6, q2)
        return 0

    lax.fori_loop(0, NCC // 4, count_slot, 0)
    for q in range(4):
        wait_cscatter(q)
    plsc.subcore_barrier()

    base = wid * NCH

    def start_meta3(c, q):
        eb = (base + c) * C
        pltpu.async_copy(dst_hbm.at[pl.ds(eb, C)], dstr[q], msem[q])
        pltpu.async_copy(typ_hbm.at[pl.ds(eb, C)], typr[q], msem[q])
        pltpu.async_copy(src_hbm.at[pl.ds(eb, C)], srcr[q], msem[q])

    def wait_meta3(c, q):
        eb = (base + c) * C
        pltpu.make_async_copy(dst_hbm.at[pl.ds(eb, C)], dstr[q], msem[q]).wait()
        pltpu.make_async_copy(typ_hbm.at[pl.ds(eb, C)], typr[q], msem[q]).wait()
        pltpu.make_async_copy(src_hbm.at[pl.ds(eb, C)], srcr[q], msem[q]).wait()

    def start_cgather(q):
        pltpu.async_copy(counts_sh.at[combr[q]], cntr[q], csem[q])

    def wait_cgather(q):
        pltpu.make_async_copy(counts_sh.at[combr[q]], cntr[q], csem[q]).wait()

    def start_out(c, k):
        eb = (base + c) * C
        pltpu.async_copy(nor[k], norm_hbm.at[pl.ds(eb, C)], wsem[k])
        pltpu.async_copy(gor[k], gidx_hbm.at[pl.ds(eb, C)], wsem[k])

    def wait_out(c, k):
        eb = (base + c) * C
        pltpu.make_async_copy(nor[k], norm_hbm.at[pl.ds(eb, C)], wsem[k]).wait()
        pltpu.make_async_copy(gor[k], gidx_hbm.at[pl.ds(eb, C)], wsem[k]).wait()

    for c in range(4):
        start_meta3(c, c)
    for c in range(2):
        wait_meta3(c, c)
        comb_compute(c)
        start_cgather(c)

    def norm_slot(i, _):
        for j in range(4):
            c = 4 * i + j
            q = j
            q2 = (j + 2) % 4
            k = j % 2

            @pl.when(c + 2 < NCH)
            def _():
                wait_meta3(c + 2, q2)
                comb_compute(q2)
                start_cgather(q2)

            wait_cgather(q)

            @pl.when(c >= 2)
            def _():
                wait_out(c - 2, k)

            for g in range(C // L):
                gl = pl.ds(g * L, L)
                nv = 1.0 / jnp.maximum(cntr[q][gl], 1.0)
                nor[k][gl] = jnp.where(dstr[q][gl] == N, 0.0, nv)
                gor[k][gl] = srcr[q][gl] * (R + 1) + typr[q][gl]
            start_out(c, k)

            @pl.when(c + 4 < NCH)
            def _():
                start_meta3(c + 4, q)
        return 0

    lax.fori_loop(0, NCH // 4, norm_slot, 0)
    for c in (NCH - 2, NCH - 1):
        wait_out(c, c % 2)


@functools.partial(
    pl.kernel,
    out_type=jax.ShapeDtypeStruct((NC, N, D), f32),
    mesh=_mesh,
    compiler_params=_params,
    scratch_types=[
        pltpu.VMEM_SHARED((N + 8, D), f32),
        [pltpu.VMEM((C, D), f32)] * 2,
        [pltpu.VMEM((C,), i32)] * 4,
        [pltpu.VMEM((C,), i32)] * 4,
        [pltpu.VMEM((C,), f32)] * 4,
        [pltpu.SemaphoreType.DMA] * 4,
        [pltpu.SemaphoreType.DMA] * 2,
        [pltpu.SemaphoreType.DMA] * 2,
    ],
)
def _layer_kernel(y_hbm, gidx_hbm, dst_hbm, norm_hbm, out_hbm, agg_sh,
                  rows, gr, dr, nr, msem, gsem, ssem):
    cid = lax.axis_index("c")
    sid = lax.axis_index("s")
    wid = _worker_id()

    def fill(i, _):
        for d in range(D // L):
            rows[0][i, pl.ds(d * L, L)] = jnp.zeros((L,), f32)
        return 0

    lax.fori_loop(0, RU, fill, 0)

    n_units = jnp.where(sid < NU - (NU // NS) * NS, NU // NS + 1, NU // NS)

    def zero_agg(u, _):
        pltpu.sync_copy(rows[0].at[pl.ds(0, RU)],
                        agg_sh.at[pl.ds((sid + u * NS) * RU, RU)])
        return 0

    lax.fori_loop(0, n_units, zero_agg, 0)
    plsc.subcore_barrier()

    base = wid * NCH

    def start_meta(c, q):
        eb = (base + c) * C
        pltpu.async_copy(gidx_hbm.at[pl.ds(eb, C)], gr[q], msem[q])
        pltpu.async_copy(dst_hbm.at[pl.ds(eb, C)], dr[q], msem[q])
        pltpu.async_copy(norm_hbm.at[pl.ds(eb, C)], nr[q], msem[q])

    def wait_meta(c, q):
        eb = (base + c) * C
        pltpu.make_async_copy(gidx_hbm.at[pl.ds(eb, C)], gr[q], msem[q]).wait()
        pltpu.make_async_copy(dst_hbm.at[pl.ds(eb, C)], dr[q], msem[q]).wait()
        pltpu.make_async_copy(norm_hbm.at[pl.ds(eb, C)], nr[q], msem[q]).wait()

    def start_gather(q, k):
        pltpu.async_copy(y_hbm.at[gr[q]], rows[k], gsem[k])

    def wait_gather(q, k):
        pltpu.make_async_copy(y_hbm.at[gr[q]], rows[k], gsem[k]).wait()

    def start_scatter(q, k):
        pltpu.async_copy(rows[k], agg_sh.at[dr[q]], ssem[k], add=True)

    def wait_scatter(q, k):
        pltpu.make_async_copy(rows[k], agg_sh.at[dr[q]], ssem[k]).wait()

    def scale(q, k):
        def body(g, _):
            nv16 = nr[q][pl.ds(g * L, L)]
            for j in range(L):
                bv = jnp.full((L,), nv16[j], f32)
                for d in range(D // L):
                    sl = pl.ds(d * L, L)
                    rows[k][g * L + j, sl] = rows[k][g * L + j, sl] * bv
            return 0

        lax.fori_loop(0, C // L, body, 0)

    for c in range(3):
        start_meta(c, c)
    wait_meta(0, 0)
    start_gather(0, 0)

    def slot(i, _):
        for j in range(4):
            c = 4 * i + j
            k = j % 2
            kn = (j + 1) % 2
            q = j
            qn = (j + 1) % 4
            qp = (j + 3) % 4

            @pl.when(c + 1 < NCH)
            def _():
                @pl.when(c >= 1)
                def _():
                    wait_scatter(qp, kn)

                wait_meta(c + 1, qn)
                start_gather(qn, kn)

                @pl.when(c + 3 < NCH)
                def _():
                    start_meta(c + 3, qp)

            @pl.when(c < NCH)
            def _():
                wait_gather(q, k)
                scale(q, k)
                start_scatter(q, k)
        return 0

    lax.fori_loop(0, NCH // 4, slot, 0)
    wait_scatter((NCH - 2) % 4, (NCH - 2) % 2)
    wait_scatter((NCH - 1) % 4, (NCH - 1) % 2)
    plsc.subcore_barrier()

    def writeback(u, _):
        rbase = (sid + u * NS) * RU
        pltpu.sync_copy(agg_sh.at[pl.ds(rbase, RU)], rows[0].at[pl.ds(0, RU)])
        pltpu.sync_copy(rows[0].at[pl.ds(0, RU)],
                        out_hbm.at[cid, pl.ds(rbase, RU)])
        return 0

    lax.fori_loop(0, n_units, writeback, 0)


@functools.partial(
    pl.kernel,
    out_type=jax.ShapeDtypeStruct((EP,), f32),
    mesh=_mesh,
    compiler_params=_params,
    scratch_types=[
        pltpu.VMEM((R, D), f32),
        [pltpu.VMEM((C, D), f32)] * 2,
        [pltpu.VMEM((C, D), f32)] * 2,
        [pltpu.VMEM((C,), i32)] * 4,
        [pltpu.VMEM((C,), i32)] * 4,
        [pltpu.VMEM((C,), i32)] * 4,
        [pltpu.VMEM((C,), f32)] * 2,
        pltpu.VMEM((L * L,), f32),
        [pltpu.SemaphoreType.DMA] * 4,
        [pltpu.SemaphoreType.DMA] * 2,
        [pltpu.SemaphoreType.DMA] * 2,
        [pltpu.SemaphoreType.DMA] * 2,
    ],
)
def _score_kernel(h_hbm, src_hbm, dst_hbm, typ_hbm, rel_hbm, out_hbm, rel_v,
                  head, tail, srcr, dstr, typr, scr, tp_v, msem, hsem, tsem,
                  wsem):
    wid = _worker_id()
    pltpu.sync_copy(rel_hbm, rel_v)
    base = wid * NCH

    def start_meta(c, q):
        eb = (base + c) * C
        pltpu.async_copy(src_hbm.at[pl.ds(eb, C)], srcr[q], msem[q])
        pltpu.async_copy(dst_hbm.at[pl.ds(eb, C)], dstr[q], msem[q])
        pltpu.async_copy(typ_hbm.at[pl.ds(eb, C)], typr[q], msem[q])

    def wait_meta(c, q):
        eb = (base + c) * C
        pltpu.make_async_copy(src_hbm.at[pl.ds(eb, C)], srcr[q], msem[q]).wait()
        pltpu.make_async_copy(dst_hbm.at[pl.ds(eb, C)], dstr[q], msem[q]).wait()
        pltpu.make_async_copy(typ_hbm.at[pl.ds(eb, C)], typr[q], msem[q]).wait()

    def start_gathers(q, k):
        pltpu.async_copy(h_hbm.at[srcr[q]], head[k], hsem[k])
        pltpu.async_copy(h_hbm.at[dstr[q]], tail[k], tsem[k])

    def wait_gathers(q, k):
        pltpu.make_async_copy(h_hbm.at[srcr[q]], head[k], hsem[k]).wait()
        pltpu.make_async_copy(h_hbm.at[dstr[q]], tail[k], tsem[k]).wait()

    def start_out(c, k):
        eb = (base + c) * C
        pltpu.async_copy(scr[k], out_hbm.at[pl.ds(eb, C)], wsem[k])

    def wait_out(c, k):
        eb = (base + c) * C
        pltpu.make_async_copy(scr[k], out_hbm.at[pl.ds(eb, C)], wsem[k]).wait()

    lane = lax.iota(i32, L)

    def compute(q, k):
        def grp(g, _):
            tv16 = typr[q][pl.ds(g * L, L)]
            for j in range(L):
                e = g * L + j
                te = tv16[j]
                acc = jnp.zeros((L,), f32)
                for d in range(D // L):
                    sl = pl.ds(d * L, L)
                    acc = acc + head[k][e, sl] * rel_v[te, sl] * tail[k][e, sl]
                plsc.store_scatter(tp_v, [lane * L + j], acc)
            res = jnp.zeros((L,), f32)
            for l in range(L):
                res = res + tp_v[pl.ds(l * L, L)]
            scr[k][pl.ds(g * L, L)] = res
            return 0

        lax.fori_loop(0, C // L, grp, 0)

    for c in range(3):
        start_meta(c, c)
    wait_meta(0, 0)
    start_gathers(0, 0)

    def slot(i, _):
        for j in range(4):
            c = 4 * i + j
            k = j % 2
            kn = (j + 1) % 2
            q = j
            qn = (j + 1) % 4
            qp = (j + 3) % 4

            @pl.when(c + 1 < NCH)
            def _():
                wait_meta(c + 1, qn)
                start_gathers(qn, kn)

                @pl.when(c + 3 < NCH)
                def _():
                    start_meta(c + 3, qp)

            @pl.when(c < NCH)
            def _():
                wait_gathers(q, k)

                @pl.when(c >= 2)
                def _():
                    wait_out(c - 2, k)

                compute(q, k)
                start_out(c, k)
        return 0

    lax.fori_loop(0, NCH // 4, slot, 0)
    for c in (NCH - 2, NCH - 1):
        wait_out(c, c % 2)


BN = 2000
BNC = 1000
DW = (R + 1) * D


def _mm_body(x_ref, w_ref, o_ref):
    o_ref[...] = jnp.dot(x_ref[...], w_ref[...], preferred_element_type=f32,
                         precision=lax.Precision.HIGHEST)


_mm = pl.pallas_call(
    _mm_body,
    grid=(N // BN,),
    in_specs=[
        pl.BlockSpec((BN, D), lambda nb: (nb, 0)),
        pl.BlockSpec((D, DW), lambda nb: (0, 0)),
    ],
    out_specs=pl.BlockSpec((BN, DW), lambda nb: (nb, 0)),
    out_shape=jax.ShapeDtypeStruct((N, DW), f32),
)


def _mmc_body(p_ref, y_ref, b_ref, w_ref, o_ref):
    h = jnp.maximum(p_ref[0] + p_ref[1] + y_ref[...] + b_ref[...], 0.0)
    o_ref[...] = jnp.dot(h, w_ref[...], preferred_element_type=f32,
                         precision=lax.Precision.HIGHEST)


_mmc = pl.pallas_call(
    _mmc_body,
    grid=(N // BN,),
    in_specs=[
        pl.BlockSpec((2, BN, D), lambda nb: (0, nb, 0)),
        pl.BlockSpec((BN, D), lambda nb: (nb, R)),
        pl.BlockSpec((1, D), lambda nb: (0, 0)),
        pl.BlockSpec((D, DW), lambda nb: (0, 0)),
    ],
    out_specs=pl.BlockSpec((BN, DW), lambda nb: (nb, 0)),
    out_shape=jax.ShapeDtypeStruct((N, DW), f32),
)


def _comb_body(p_ref, y_ref, b_ref, o_ref):
    o_ref[...] = jnp.maximum(p_ref[0] + p_ref[1] + y_ref[...] + b_ref[...], 0.0)


_comb = pl.pallas_call(
    _comb_body,
    grid=(N // BNC,),
    in_specs=[
        pl.BlockSpec((2, BNC, D), lambda nb: (0, nb, 0)),
        pl.BlockSpec((BNC, D), lambda nb: (nb, R)),
        pl.BlockSpec((1, D), lambda nb: (0, 0)),
    ],
    out_specs=pl.BlockSpec((BNC, D), lambda nb: (nb, 0)),
    out_shape=jax.ShapeDtypeStruct((N, D), f32),
)


def _wcat(W, root):
    return jnp.concatenate(
        [W.transpose(1, 0, 2).reshape(D, R * D), root], axis=1)


def kernel(x, edge_index, edge_type, W1, root1, b1, W2, root2, b2, rel_emb):
    pad = EP - E
    spread = jnp.arange(pad, dtype=i32) % N
    srcp = jnp.concatenate([edge_index[0], spread])
    dstp = jnp.concatenate([edge_index[1], spread])
    typp = jnp.concatenate([edge_type, jnp.arange(pad, dtype=i32) % R])
    dst_cnt = jnp.concatenate([edge_index[1], jnp.full((pad,), N, i32)])

    norm, gidx = _norm_kernel(dst_cnt, typp, srcp)

    y1 = _mm(x, _wcat(W1, root1))
    p1 = _layer_kernel(y1.reshape(N * (R + 1), D), gidx, dstp, norm)

    y2 = _mmc(p1, y1, b1.reshape(1, D), _wcat(W2, root2))
    p2 = _layer_kernel(y2.reshape(N * (R + 1), D), gidx, dstp, norm)
    h2 = _comb(p2, y2, b2.reshape(1, D))

    return _score_kernel(h2, srcp, dstp, typp, rel_emb)[:E]


# speedup vs baseline: 39.1839x; 1.1105x over previous
import functools

import jax
import jax.numpy as jnp
from jax import lax
from jax.experimental import pallas as pl
from jax.experimental.pallas import tpu as pltpu
from jax.experimental.pallas import tpu_sc as plsc

N = 10000
R = 16
D = 128
E = 320000

NC = 2
NS = 16
L = 16
NW = NC * NS
C = 128
EP = 327680
NCH = EP // NW // C
NCC = EP // NS // C
NR2 = 161280
ZS = NR2 // NS
RU = 80
NU = N // RU

f32 = jnp.float32
i32 = jnp.int32

_mesh = plsc.VectorSubcoreMesh(core_axis_name="c", subcore_axis_name="s")
_params = pltpu.CompilerParams(needs_layout_passes=False)


def _worker_id():
    return lax.axis_index("s") * NC + lax.axis_index("c")


@functools.partial(
    pl.kernel,
    out_type=[
        jax.ShapeDtypeStruct((EP,), f32),
        jax.ShapeDtypeStruct((EP,), i32),
    ],
    mesh=_mesh,
    compiler_params=_params,
    scratch_types=[
        pltpu.VMEM_SHARED((NR2,), f32),
        pltpu.VMEM((2048,), f32),
        [pltpu.VMEM((C,), i32)] * 4,
        [pltpu.VMEM((C,), i32)] * 4,
        [pltpu.VMEM((C,), i32)] * 4,
        [pltpu.VMEM((C,), i32)] * 4,
        [pltpu.VMEM((C,), f32)] * 4,
        [pltpu.VMEM((C,), f32)] * 2,
        [pltpu.VMEM((C,), i32)] * 2,
        [pltpu.SemaphoreType.DMA] * 4,
        [pltpu.SemaphoreType.DMA] * 4,
        [pltpu.SemaphoreType.DMA] * 2,
    ],
)
def _norm_kernel(dst_hbm, typ_hbm, src_hbm, norm_hbm, gidx_hbm, counts_sh,
                 stage_v, dstr, typr, srcr, combr, cntr, nor, gor, msem, csem,
                 wsem):
    sid = lax.axis_index("s")
    wid = _worker_id()

    def fill(i, _):
        stage_v[pl.ds(i * L, L)] = jnp.zeros((L,), f32)
        return 0

    lax.fori_loop(0, 2048 // L, fill, 0)

    def zero_counts(j, _):
        pltpu.sync_copy(stage_v.at[pl.ds(0, 2016)],
                        counts_sh.at[pl.ds(sid * ZS + j * 2016, 2016)])
        return 0

    lax.fori_loop(0, ZS // 2016, zero_counts, 0)
    plsc.subcore_barrier()

    for g in range(C // L):
        stage_v[pl.ds(g * L, L)] = jnp.ones((L,), f32)
    ones = stage_v.at[pl.ds(0, C)]

    cbase = sid * NCC

    def start_meta2(c, q):
        eb = (cbase + c) * C
        pltpu.async_copy(dst_hbm.at[pl.ds(eb, C)], dstr[q], msem[q])
        pltpu.async_copy(typ_hbm.at[pl.ds(eb, C)], typr[q], msem[q])

    def wait_meta2(c, q):
        eb = (cbase + c) * C
        pltpu.make_async_copy(dst_hbm.at[pl.ds(eb, C)], dstr[q], msem[q]).wait()
        pltpu.make_async_copy(typ_hbm.at[pl.ds(eb, C)], typr[q], msem[q]).wait()

    def comb_compute(q):
        for g in range(C // L):
            gl = pl.ds(g * L, L)
            combr[q][gl] = dstr[q][gl] * R + typr[q][gl]

    def start_cscatter(q):
        pltpu.async_copy(ones, counts_sh.at[combr[q]], csem[q], add=True)

    def wait_cscatter(q):
        pltpu.make_async_copy(ones, counts_sh.at[combr[q]], csem[q]).wait()

    for c in range(4):
        start_meta2(c, c)
    for c in range(2):
        wait_meta2(c, c)
        comb_compute(c)
        start_cscatter(c)
    start_meta2(4, 0)
    start_meta2(5, 1)

    def count_slot(i, _):
        for j in range(4):
            c = 4 * i + j
            q2 = (j + 2) % 4

            @pl.when(c + 2 < NCC)
            def _():
                @pl.when(c >= 2)
                def _():
                    wait_cscatter(q2)

                wait_meta2(c + 2, q2)
                comb_compute(q2)
                start_cscatter(q2)

                @pl.when(c + 6 < NCC)
                def _():
                    start_meta2(c + 6, q2)
        return 0

    lax.fori_loop(0, NCC // 4, count_slot, 0)
    for q in range(4):
        wait_cscatter(q)
    plsc.subcore_barrier()

    base = wid * NCH

    def start_meta3(c, q):
        eb = (base + c) * C
        pltpu.async_copy(dst_hbm.at[pl.ds(eb, C)], dstr[q], msem[q])
        pltpu.async_copy(typ_hbm.at[pl.ds(eb, C)], typr[q], msem[q])
        pltpu.async_copy(src_hbm.at[pl.ds(eb, C)], srcr[q], msem[q])

    def wait_meta3(c, q):
        eb = (base + c) * C
        pltpu.make_async_copy(dst_hbm.at[pl.ds(eb, C)], dstr[q], msem[q]).wait()
        pltpu.make_async_copy(typ_hbm.at[pl.ds(eb, C)], typr[q], msem[q]).wait()
        pltpu.make_async_copy(src_hbm.at[pl.ds(eb, C)], srcr[q], msem[q]).wait()

    def start_cgather(q):
        pltpu.async_copy(counts_sh.at[combr[q]], cntr[q], csem[q])

    def wait_cgather(q):
        pltpu.make_async_copy(counts_sh.at[combr[q]], cntr[q], csem[q]).wait()

    def start_out(c, k):
        eb = (base + c) * C
        pltpu.async_copy(nor[k], norm_hbm.at[pl.ds(eb, C)], wsem[k])
        pltpu.async_copy(gor[k], gidx_hbm.at[pl.ds(eb, C)], wsem[k])

    def wait_out(c, k):
        eb = (base + c) * C
        pltpu.make_async_copy(nor[k], norm_hbm.at[pl.ds(eb, C)], wsem[k]).wait()
        pltpu.make_async_copy(gor[k], gidx_hbm.at[pl.ds(eb, C)], wsem[k]).wait()

    for c in range(4):
        start_meta3(c, c)
    for c in range(2):
        wait_meta3(c, c)
        comb_compute(c)
        start_cgather(c)

    def norm_slot(i, _):
        for j in range(4):
            c = 4 * i + j
            q = j
            q2 = (j + 2) % 4
            k = j % 2

            @pl.when(c + 2 < NCH)
            def _():
                wait_meta3(c + 2, q2)
                comb_compute(q2)
                start_cgather(q2)

            wait_cgather(q)

            @pl.when(c >= 2)
            def _():
                wait_out(c - 2, k)

            for g in range(C // L):
                gl = pl.ds(g * L, L)
                nv = 1.0 / jnp.maximum(cntr[q][gl], 1.0)
                nor[k][gl] = jnp.where(dstr[q][gl] == N, 0.0, nv)
                gor[k][gl] = srcr[q][gl] * (R + 1) + typr[q][gl]
            start_out(c, k)

            @pl.when(c + 4 < NCH)
            def _():
                start_meta3(c + 4, q)
        return 0

    lax.fori_loop(0, NCH // 4, norm_slot, 0)
    for c in (NCH - 2, NCH - 1):
        wait_out(c, c % 2)


@functools.partial(
    pl.kernel,
    out_type=jax.ShapeDtypeStruct((NC, N, D), f32),
    mesh=_mesh,
    compiler_params=_params,
    scratch_types=[
        pltpu.VMEM_SHARED((N + 8, D), f32),
        [pltpu.VMEM((C, D), f32)] * 2,
        [pltpu.VMEM((C,), i32)] * 4,
        [pltpu.VMEM((C,), i32)] * 4,
        [pltpu.VMEM((C,), f32)] * 4,
        [pltpu.SemaphoreType.DMA] * 4,
        [pltpu.SemaphoreType.DMA] * 2,
        [pltpu.SemaphoreType.DMA] * 2,
    ],
)
def _layer_kernel(y_hbm, gidx_hbm, dst_hbm, norm_hbm, out_hbm, agg_sh,
                  rows, gr, dr, nr, msem, gsem, ssem):
    cid = lax.axis_index("c")
    sid = lax.axis_index("s")
    wid = _worker_id()

    def fill(i, _):
        for d in range(D // L):
            rows[0][i, pl.ds(d * L, L)] = jnp.zeros((L,), f32)
        return 0

    lax.fori_loop(0, RU, fill, 0)

    n_units = jnp.where(sid < NU - (NU // NS) * NS, NU // NS + 1, NU // NS)

    def zero_agg(u, _):
        pltpu.sync_copy(rows[0].at[pl.ds(0, RU)],
                        agg_sh.at[pl.ds((sid + u * NS) * RU, RU)])
        return 0

    lax.fori_loop(0, n_units, zero_agg, 0)
    plsc.subcore_barrier()

    base = wid * NCH

    def start_meta(c, q):
        eb = (base + c) * C
        pltpu.async_copy(gidx_hbm.at[pl.ds(eb, C)], gr[q], msem[q])
        pltpu.async_copy(dst_hbm.at[pl.ds(eb, C)], dr[q], msem[q])
        pltpu.async_copy(norm_hbm.at[pl.ds(eb, C)], nr[q], msem[q])

    def wait_meta(c, q):
        eb = (base + c) * C
        pltpu.make_async_copy(gidx_hbm.at[pl.ds(eb, C)], gr[q], msem[q]).wait()
        pltpu.make_async_copy(dst_hbm.at[pl.ds(eb, C)], dr[q], msem[q]).wait()
        pltpu.make_async_copy(norm_hbm.at[pl.ds(eb, C)], nr[q], msem[q]).wait()

    def start_gather(q, k):
        pltpu.async_copy(y_hbm.at[gr[q]], rows[k], gsem[k])

    def wait_gather(q, k):
        pltpu.make_async_copy(y_hbm.at[gr[q]], rows[k], gsem[k]).wait()

    def start_scatter(q, k):
        pltpu.async_copy(rows[k], agg_sh.at[dr[q]], ssem[k], add=True)

    def wait_scatter(q, k):
        pltpu.make_async_copy(rows[k], agg_sh.at[dr[q]], ssem[k]).wait()

    def scale(q, k):
        def body(g, _):
            nv16 = nr[q][pl.ds(g * L, L)]
            for j in range(L):
                bv = jnp.full((L,), nv16[j], f32)
                for d in range(D // L):
                    sl = pl.ds(d * L, L)
                    rows[k][g * L + j, sl] = rows[k][g * L + j, sl] * bv
            return 0

        lax.fori_loop(0, C // L, body, 0)

    for c in range(3):
        start_meta(c, c)
    wait_meta(0, 0)
    start_gather(0, 0)

    def slot(i, _):
        for j in range(4):
            c = 4 * i + j
            k = j % 2
            kn = (j + 1) % 2
            q = j
            qn = (j + 1) % 4
            qp = (j + 3) % 4

            @pl.when(c + 1 < NCH)
            def _():
                @pl.when(c >= 1)
                def _():
                    wait_scatter(qp, kn)

                wait_meta(c + 1, qn)
                start_gather(qn, kn)

                @pl.when(c + 3 < NCH)
                def _():
                    start_meta(c + 3, qp)

            @pl.when(c < NCH)
            def _():
                wait_gather(q, k)
                scale(q, k)
                start_scatter(q, k)
        return 0

    lax.fori_loop(0, NCH // 4, slot, 0)
    wait_scatter((NCH - 2) % 4, (NCH - 2) % 2)
    wait_scatter((NCH - 1) % 4, (NCH - 1) % 2)
    plsc.subcore_barrier()

    def writeback(u, _):
        rbase = (sid + u * NS) * RU
        pltpu.sync_copy(agg_sh.at[pl.ds(rbase, RU)], rows[0].at[pl.ds(0, RU)])
        pltpu.sync_copy(rows[0].at[pl.ds(0, RU)],
                        out_hbm.at[cid, pl.ds(rbase, RU)])
        return 0

    lax.fori_loop(0, n_units, writeback, 0)


@functools.partial(
    pl.kernel,
    out_type=jax.ShapeDtypeStruct((EP,), f32),
    mesh=_mesh,
    compiler_params=_params,
    scratch_types=[
        pltpu.VMEM((R, D), f32),
        [pltpu.VMEM((C, D), f32)] * 2,
        [pltpu.VMEM((C, D), f32)] * 2,
        [pltpu.VMEM((C,), i32)] * 4,
        [pltpu.VMEM((C,), i32)] * 4,
        [pltpu.VMEM((C,), i32)] * 4,
        [pltpu.VMEM((C,), f32)] * 2,
        pltpu.VMEM((L * L,), f32),
        [pltpu.SemaphoreType.DMA] * 4,
        [pltpu.SemaphoreType.DMA] * 2,
        [pltpu.SemaphoreType.DMA] * 2,
        [pltpu.SemaphoreType.DMA] * 2,
    ],
)
def _score_kernel(h_hbm, src_hbm, dst_hbm, typ_hbm, rel_hbm, out_hbm, rel_v,
                  head, tail, srcr, dstr, typr, scr, tp_v, msem, hsem, tsem,
                  wsem):
    wid = _worker_id()
    pltpu.sync_copy(rel_hbm, rel_v)
    base = wid * NCH

    def start_meta(c, q):
        eb = (base + c) * C
        pltpu.async_copy(src_hbm.at[pl.ds(eb, C)], srcr[q], msem[q])
        pltpu.async_copy(dst_hbm.at[pl.ds(eb, C)], dstr[q], msem[q])
        pltpu.async_copy(typ_hbm.at[pl.ds(eb, C)], typr[q], msem[q])

    def wait_meta(c, q):
        eb = (base + c) * C
        pltpu.make_async_copy(src_hbm.at[pl.ds(eb, C)], srcr[q], msem[q]).wait()
        pltpu.make_async_copy(dst_hbm.at[pl.ds(eb, C)], dstr[q], msem[q]).wait()
        pltpu.make_async_copy(typ_hbm.at[pl.ds(eb, C)], typr[q], msem[q]).wait()

    def start_gathers(q, k):
        pltpu.async_copy(h_hbm.at[srcr[q]], head[k], hsem[k])
        pltpu.async_copy(h_hbm.at[dstr[q]], tail[k], tsem[k])

    def wait_gathers(q, k):
        pltpu.make_async_copy(h_hbm.at[srcr[q]], head[k], hsem[k]).wait()
        pltpu.make_async_copy(h_hbm.at[dstr[q]], tail[k], tsem[k]).wait()

    def start_out(c, k):
        eb = (base + c) * C
        pltpu.async_copy(scr[k], out_hbm.at[pl.ds(eb, C)], wsem[k])

    def wait_out(c, k):
        eb = (base + c) * C
        pltpu.make_async_copy(scr[k], out_hbm.at[pl.ds(eb, C)], wsem[k]).wait()

    lane = lax.iota(i32, L)

    def compute(q, k):
        def grp(g, _):
            tv16 = typr[q][pl.ds(g * L, L)]
            for j in range(L):
                e = g * L + j
                te = tv16[j]
                acc = jnp.zeros((L,), f32)
                for d in range(D // L):
                    sl = pl.ds(d * L, L)
                    acc = acc + head[k][e, sl] * rel_v[te, sl] * tail[k][e, sl]
                plsc.store_scatter(tp_v, [lane * L + j], acc)
            res = jnp.zeros((L,), f32)
            for l in range(L):
                res = res + tp_v[pl.ds(l * L, L)]
            scr[k][pl.ds(g * L, L)] = res
            return 0

        lax.fori_loop(0, C // L, grp, 0)

    for c in range(3):
        start_meta(c, c)
    wait_meta(0, 0)
    start_gathers(0, 0)

    def slot(i, _):
        for j in range(4):
            c = 4 * i + j
            k = j % 2
            kn = (j + 1) % 2
            q = j
            qn = (j + 1) % 4
            qp = (j + 3) % 4

            @pl.when(c + 1 < NCH)
            def _():
                wait_meta(c + 1, qn)
                start_gathers(qn, kn)

                @pl.when(c + 3 < NCH)
                def _():
                    start_meta(c + 3, qp)

            @pl.when(c < NCH)
            def _():
                wait_gathers(q, k)

                @pl.when(c >= 2)
                def _():
                    wait_out(c - 2, k)

                compute(q, k)
                start_out(c, k)
        return 0

    lax.fori_loop(0, NCH // 4, slot, 0)
    for c in (NCH - 2, NCH - 1):
        wait_out(c, c % 2)


BN = 2000
BNC = 1000
DW = (R + 1) * D


def _mm_body(x_ref, w_ref, o_ref):
    o_ref[...] = jnp.dot(x_ref[...], w_ref[...], preferred_element_type=f32,
                         precision=lax.Precision.DEFAULT)


_mm = pl.pallas_call(
    _mm_body,
    grid=(N // BN,),
    in_specs=[
        pl.BlockSpec((BN, D), lambda nb: (nb, 0)),
        pl.BlockSpec((D, DW), lambda nb: (0, 0)),
    ],
    out_specs=pl.BlockSpec((BN, DW), lambda nb: (nb, 0)),
    out_shape=jax.ShapeDtypeStruct((N, DW), f32),
)


def _mmc_body(p_ref, y_ref, b_ref, w_ref, o_ref):
    h = jnp.maximum(p_ref[0] + p_ref[1] + y_ref[...] + b_ref[...], 0.0)
    o_ref[...] = jnp.dot(h, w_ref[...], preferred_element_type=f32,
                         precision=lax.Precision.DEFAULT)


_mmc = pl.pallas_call(
    _mmc_body,
    grid=(N // BN,),
    in_specs=[
        pl.BlockSpec((2, BN, D), lambda nb: (0, nb, 0)),
        pl.BlockSpec((BN, D), lambda nb: (nb, R)),
        pl.BlockSpec((1, D), lambda nb: (0, 0)),
        pl.BlockSpec((D, DW), lambda nb: (0, 0)),
    ],
    out_specs=pl.BlockSpec((BN, DW), lambda nb: (nb, 0)),
    out_shape=jax.ShapeDtypeStruct((N, DW), f32),
)


def _comb_body(p_ref, y_ref, b_ref, o_ref):
    o_ref[...] = jnp.maximum(p_ref[0] + p_ref[1] + y_ref[...] + b_ref[...], 0.0)


_comb = pl.pallas_call(
    _comb_body,
    grid=(N // BNC,),
    in_specs=[
        pl.BlockSpec((2, BNC, D), lambda nb: (0, nb, 0)),
        pl.BlockSpec((BNC, D), lambda nb: (nb, R)),
        pl.BlockSpec((1, D), lambda nb: (0, 0)),
    ],
    out_specs=pl.BlockSpec((BNC, D), lambda nb: (nb, 0)),
    out_shape=jax.ShapeDtypeStruct((N, D), f32),
)


def _wcat(W, root):
    return jnp.concatenate(
        [W.transpose(1, 0, 2).reshape(D, R * D), root], axis=1)


def kernel(x, edge_index, edge_type, W1, root1, b1, W2, root2, b2, rel_emb):
    pad = EP - E
    spread = jnp.arange(pad, dtype=i32) % N
    srcp = jnp.concatenate([edge_index[0], spread])
    dstp = jnp.concatenate([edge_index[1], spread])
    typp = jnp.concatenate([edge_type, jnp.arange(pad, dtype=i32) % R])
    dst_cnt = jnp.concatenate([edge_index[1], jnp.full((pad,), N, i32)])

    norm, gidx = _norm_kernel(dst_cnt, typp, srcp)

    y1 = _mm(x, _wcat(W1, root1))
    p1 = _layer_kernel(y1.reshape(N * (R + 1), D), gidx, dstp, norm)

    y2 = _mmc(p1, y1, b1.reshape(1, D), _wcat(W2, root2))
    p2 = _layer_kernel(y2.reshape(N * (R + 1), D), gidx, dstp, norm)
    h2 = _comb(p2, y2, b2.reshape(1, D))

    return _score_kernel(h2, srcp, dstp, typp, rel_emb)[:E]
